# baseline (speedup 1.0000x reference)
import functools

import jax
import jax.numpy as jnp
from jax.experimental import pallas as pl
from jax.experimental.pallas import tpu as pltpu

_VMEM = 48 * 1024 * 1024
_COUNTS = (3, 4, 6, 3)


def _round_up(x, m):
    return (x + m - 1) // m * m


def _pick_tm(M):
    for tm in (512, 448, 784, 256, 128, 64, 32, 16, 8):
        if M % tm == 0:
            return tm
    return 256


def _cparams(*sem):
    return pltpu.CompilerParams(dimension_semantics=sem,
                                vmem_limit_bytes=_VMEM)


def _mm_kernel(a_ref, w_ref, b_ref, o_ref, *, relu):
    acc = jnp.dot(a_ref[...], w_ref[...], preferred_element_type=jnp.float32)
    acc = acc + b_ref[...]
    if relu:
        acc = jnp.maximum(acc, 0.0)
    o_ref[...] = acc.astype(o_ref.dtype)


def _mm_res_kernel(a_ref, w_ref, b_ref, r_ref, o_ref, *, relu):
    acc = jnp.dot(a_ref[...], w_ref[...], preferred_element_type=jnp.float32)
    acc = acc + b_ref[...] + r_ref[...].astype(jnp.float32)
    if relu:
        acc = jnp.maximum(acc, 0.0)
    o_ref[...] = acc.astype(o_ref.dtype)


def _gconv_kernel(a_ref, w_ref, b_ref, o_ref):
    acc = jnp.dot(a_ref[...], w_ref[0], preferred_element_type=jnp.float32)
    acc = jnp.maximum(acc + b_ref[0], 0.0)
    o_ref[...] = acc.astype(o_ref.dtype)


def _max_kernel(x_ref, o_ref):
    o_ref[...] = jnp.max(x_ref[...], axis=1)


def _avg_kernel(x_ref, o_ref):
    x = x_ref[...].astype(jnp.float32)
    o_ref[...] = jnp.sum(x, axis=1) * (1.0 / x.shape[1])


def _matmul(a, w, bias, residual=None, *, relu):
    M, K = a.shape
    N = w.shape[1]
    tm = _pick_tm(M)
    Mp = _round_up(M, tm)
    if Mp != M:
        a = jnp.pad(a, ((0, Mp - M), (0, 0)))
    tn = 512 if N % 512 == 0 else (256 if N % 256 == 0 else 128)
    bias2 = bias.reshape(1, N)
    grid = (Mp // tm, N // tn)
    in_specs = [
        pl.BlockSpec((tm, K), lambda i, j: (i, 0)),
        pl.BlockSpec((K, tn), lambda i, j: (0, j)),
        pl.BlockSpec((1, tn), lambda i, j: (0, j)),
    ]
    args = [a, w, bias2]
    if residual is not None:
        r = residual
        if Mp != M:
            r = jnp.pad(r, ((0, Mp - M), (0, 0)))
        in_specs.append(pl.BlockSpec((tm, tn), lambda i, j: (i, j)))
        args.append(r)
        body = functools.partial(_mm_res_kernel, relu=relu)
    else:
        body = functools.partial(_mm_kernel, relu=relu)
    out = pl.pallas_call(
        body,
        out_shape=jax.ShapeDtypeStruct((Mp, N), jnp.bfloat16),
        grid=grid,
        in_specs=in_specs,
        out_specs=pl.BlockSpec((tm, tn), lambda i, j: (i, j)),
        compiler_params=_cparams("parallel", "parallel"),
    )(*args)
    return out[:M] if Mp != M else out


def _conv1x1(x, w, b, *, relu, stride=1, residual=None):
    if stride > 1:
        x = x[:, ::stride, ::stride, :]
    B, H, W, C = x.shape
    out = _matmul(x.reshape(B * H * W, C), w, b, residual=residual, relu=relu)
    return out.reshape(B, H, W, w.shape[1])


def _tap_views(xp, Ho, Wo, kh, kw, stride):
    views = []
    for i in range(kh):
        for j in range(kw):
            views.append(xp[:, i:i + stride * (Ho - 1) + 1:stride,
                            j:j + stride * (Wo - 1) + 1:stride, :])
    return views


def _gconv3x3(x, w2, b2, *, stride):
    B, H, W, C = x.shape
    G = C // 128
    xp = jnp.pad(x, ((0, 0), (1, 1), (1, 1), (0, 0)))
    Ho = (H + 2 - 3) // stride + 1
    Wo = (W + 2 - 3) // stride + 1
    views = _tap_views(xp, Ho, Wo, 3, 3, stride)
    M = B * Ho * Wo
    if G == 1:
        a = jnp.stack(views, axis=3).reshape(M, 9 * C)
        wk = w2.reshape(1, 9 * 128, 128)
    else:
        a = jnp.stack(views, axis=-1).reshape(M, C * 9)
        wt = w2.reshape(9, G, 128, G, 128)
        gi = jnp.arange(G)
        wd = wt[:, gi, :, gi, :]
        wk = jnp.transpose(wd, (0, 2, 1, 3)).reshape(G, 9 * 128, 128)
    bias = b2.reshape(G, 1, 128)
    tm = _pick_tm(M)
    Mp = _round_up(M, tm)
    if Mp != M:
        a = jnp.pad(a, ((0, Mp - M), (0, 0)))
    out = pl.pallas_call(
        _gconv_kernel,
        out_shape=jax.ShapeDtypeStruct((Mp, C), jnp.bfloat16),
        grid=(Mp // tm, G),
        in_specs=[
            pl.BlockSpec((tm, 9 * 128), lambda i, j: (i, j)),
            pl.BlockSpec((1, 9 * 128, 128), lambda i, j: (j, 0, 0)),
            pl.BlockSpec((1, 1, 128), lambda i, j: (j, 0, 0)),
        ],
        out_specs=pl.BlockSpec((tm, 128), lambda i, j: (i, j)),
        compiler_params=_cparams("parallel", "parallel"),
    )(a, wk, bias)
    if Mp != M:
        out = out[:M]
    return out.reshape(B, Ho, Wo, C)


def _stem(x_nchw, w, b):
    x = jnp.transpose(x_nchw, (0, 2, 3, 1)).astype(jnp.bfloat16)
    B, H, W, C = x.shape
    xp = jnp.pad(x, ((0, 0), (3, 3), (3, 3), (0, 0)))
    Ho = (H + 6 - 7) // 2 + 1
    Wo = (W + 6 - 7) // 2 + 1
    views = _tap_views(xp, Ho, Wo, 7, 7, 2)
    M = B * Ho * Wo
    a = jnp.stack(views, axis=3).reshape(M, 49 * C)
    a = jnp.pad(a, ((0, 0), (0, w.shape[0] - 49 * C)))
    out = _matmul(a, w, b, relu=True)
    return out.reshape(B, Ho, Wo, w.shape[1])


def _maxpool3x3s2(x):
    B, H, W, C = x.shape
    xp = jnp.pad(x, ((0, 0), (1, 1), (1, 1), (0, 0)),
                 constant_values=-jnp.inf)
    Ho = (H + 2 - 3) // 2 + 1
    Wo = (W + 2 - 3) // 2 + 1
    views = _tap_views(xp, Ho, Wo, 3, 3, 2)
    M = B * Ho * Wo
    a = jnp.stack(views, axis=3).reshape(M, 9, C)
    tm = _pick_tm(M)
    out = pl.pallas_call(
        _max_kernel,
        out_shape=jax.ShapeDtypeStruct((M, C), x.dtype),
        grid=(M // tm,),
        in_specs=[pl.BlockSpec((tm, 9, C), lambda i: (i, 0, 0))],
        out_specs=pl.BlockSpec((tm, C), lambda i: (i, 0)),
        compiler_params=_cparams("parallel"),
    )(a)
    return out.reshape(B, Ho, Wo, C)


def _avgpool(x):
    B, H, W, C = x.shape
    a = x.reshape(B, H * W, C)
    tc = 256
    out = pl.pallas_call(
        _avg_kernel,
        out_shape=jax.ShapeDtypeStruct((B, C), jnp.float32),
        grid=(C // tc,),
        in_specs=[pl.BlockSpec((B, H * W, tc), lambda j: (0, 0, j))],
        out_specs=pl.BlockSpec((B, tc), lambda j: (0, j)),
        compiler_params=_cparams("parallel"),
    )(a)
    return out


def _bottleneck(x, w1, b1, w2, b2, w3, b3, wd, bd, stride):
    out = _conv1x1(x, w1, b1, relu=True)
    out = _gconv3x3(out, w2, b2, stride=stride)
    if wd is not None:
        identity = _conv1x1(x, wd, bd, relu=False, stride=stride)
    else:
        identity = x
    B, H, W, C = out.shape
    res = identity.reshape(B * H * W, identity.shape[-1])
    out = _matmul(out.reshape(B * H * W, C), w3, b3, residual=res, relu=True)
    return out.reshape(B, H, W, w3.shape[1])


def kernel(stem_w, stem_b, s0_b0_w1, s0_b0_b1, s0_b0_w2, s0_b0_b2, s0_b0_w3, s0_b0_b3, s0_b0_wd, s0_b0_bd, s0_b1_w1, s0_b1_b1, s0_b1_w2, s0_b1_b2, s0_b1_w3, s0_b1_b3, s0_b2_w1, s0_b2_b1, s0_b2_w2, s0_b2_b2, s0_b2_w3, s0_b2_b3, s1_b0_w1, s1_b0_b1, s1_b0_w2, s1_b0_b2, s1_b0_w3, s1_b0_b3, s1_b0_wd, s1_b0_bd, s1_b1_w1, s1_b1_b1, s1_b1_w2, s1_b1_b2, s1_b1_w3, s1_b1_b3, s1_b2_w1, s1_b2_b1, s1_b2_w2, s1_b2_b2, s1_b2_w3, s1_b2_b3, s1_b3_w1, s1_b3_b1, s1_b3_w2, s1_b3_b2, s1_b3_w3, s1_b3_b3, s2_b0_w1, s2_b0_b1, s2_b0_w2, s2_b0_b2, s2_b0_w3, s2_b0_b3, s2_b0_wd, s2_b0_bd, s2_b1_w1, s2_b1_b1, s2_b1_w2, s2_b1_b2, s2_b1_w3, s2_b1_b3, s2_b2_w1, s2_b2_b1, s2_b2_w2, s2_b2_b2, s2_b2_w3, s2_b2_b3, s2_b3_w1, s2_b3_b1, s2_b3_w2, s2_b3_b2, s2_b3_w3, s2_b3_b3, s2_b4_w1, s2_b4_b1, s2_b4_w2, s2_b4_b2, s2_b4_w3, s2_b4_b3, s2_b5_w1, s2_b5_b1, s2_b5_w2, s2_b5_b2, s2_b5_w3, s2_b5_b3, s3_b0_w1, s3_b0_b1, s3_b0_w2, s3_b0_b2, s3_b0_w3, s3_b0_b3, s3_b0_wd, s3_b0_bd, s3_b1_w1, s3_b1_b1, s3_b1_w2, s3_b1_b2, s3_b1_w3, s3_b1_b3, s3_b2_w1, s3_b2_b1, s3_b2_w2, s3_b2_b2, s3_b2_w3, s3_b2_b3, x):
    L = dict(locals())
    out = _stem(x, stem_w, stem_b)
    out = _maxpool3x3s2(out)
    for si, cnt in enumerate(_COUNTS):
        for bi in range(cnt):
            stride = 2 if (bi == 0 and si > 0) else 1
            out = _bottleneck(
                out,
                L[f"s{si}_b{bi}_w1"], L[f"s{si}_b{bi}_b1"],
                L[f"s{si}_b{bi}_w2"], L[f"s{si}_b{bi}_b2"],
                L[f"s{si}_b{bi}_w3"], L[f"s{si}_b{bi}_b3"],
                L.get(f"s{si}_b{bi}_wd"), L.get(f"s{si}_b{bi}_bd"),
                stride,
            )
    pool = _avgpool(out)
    return pool.reshape(pool.shape[0], -1, 1, 1)


# speedup vs baseline: 1.7054x; 1.7054x over previous
import functools

import jax
import jax.numpy as jnp
from jax.experimental import pallas as pl
from jax.experimental.pallas import tpu as pltpu

_VMEM = 48 * 1024 * 1024
_COUNTS = (3, 4, 6, 3)


def _round_up(x, m):
    return (x + m - 1) // m * m


def _pick_tm(M):
    for tm in (512, 448, 784, 256, 128, 64, 32, 16, 8):
        if M % tm == 0:
            return tm
    return 256


def _cparams(*sem):
    return pltpu.CompilerParams(dimension_semantics=sem,
                                vmem_limit_bytes=_VMEM)


def _mm_kernel(a_ref, w_ref, b_ref, o_ref, *, relu):
    acc = jnp.dot(a_ref[...], w_ref[...], preferred_element_type=jnp.float32)
    acc = acc + b_ref[...]
    if relu:
        acc = jnp.maximum(acc, 0.0)
    o_ref[...] = acc.astype(o_ref.dtype)


def _mm_res_kernel(a_ref, w_ref, b_ref, r_ref, o_ref, *, relu):
    acc = jnp.dot(a_ref[...], w_ref[...], preferred_element_type=jnp.float32)
    acc = acc + b_ref[...] + r_ref[...].astype(jnp.float32)
    if relu:
        acc = jnp.maximum(acc, 0.0)
    o_ref[...] = acc.astype(o_ref.dtype)


def _gconv_kernel(a_ref, w_ref, b_ref, o_ref):
    a = jnp.concatenate([a_ref[t] for t in range(9)], axis=1)
    acc = jnp.dot(a, w_ref[0], preferred_element_type=jnp.float32)
    acc = jnp.maximum(acc + b_ref[0], 0.0)
    o_ref[...] = acc.astype(o_ref.dtype)


def _max_kernel(x_ref, o_ref):
    o_ref[...] = jnp.max(x_ref[...], axis=0)


def _avg_kernel(x_ref, o_ref):
    x = x_ref[...].astype(jnp.float32)
    o_ref[...] = jnp.sum(x, axis=1) * (1.0 / x.shape[1])


def _matmul(a, w, bias, residual=None, *, relu):
    M, K = a.shape
    N = w.shape[1]
    tm = _pick_tm(M)
    Mp = _round_up(M, tm)
    if Mp != M:
        a = jnp.pad(a, ((0, Mp - M), (0, 0)))
    tn = 512 if N % 512 == 0 else (256 if N % 256 == 0 else 128)
    bias2 = bias.reshape(1, N)
    grid = (Mp // tm, N // tn)
    in_specs = [
        pl.BlockSpec((tm, K), lambda i, j: (i, 0)),
        pl.BlockSpec((K, tn), lambda i, j: (0, j)),
        pl.BlockSpec((1, tn), lambda i, j: (0, j)),
    ]
    args = [a, w, bias2]
    if residual is not None:
        r = residual
        if Mp != M:
            r = jnp.pad(r, ((0, Mp - M), (0, 0)))
        in_specs.append(pl.BlockSpec((tm, tn), lambda i, j: (i, j)))
        args.append(r)
        body = functools.partial(_mm_res_kernel, relu=relu)
    else:
        body = functools.partial(_mm_kernel, relu=relu)
    out = pl.pallas_call(
        body,
        out_shape=jax.ShapeDtypeStruct((Mp, N), jnp.bfloat16),
        grid=grid,
        in_specs=in_specs,
        out_specs=pl.BlockSpec((tm, tn), lambda i, j: (i, j)),
        compiler_params=_cparams("parallel", "parallel"),
    )(*args)
    return out[:M] if Mp != M else out


def _conv1x1(x, w, b, *, relu, stride=1, residual=None):
    if stride > 1:
        x = x[:, ::stride, ::stride, :]
    B, H, W, C = x.shape
    out = _matmul(x.reshape(B * H * W, C), w, b, residual=residual, relu=relu)
    return out.reshape(B, H, W, w.shape[1])


def _tap_views(xp, Ho, Wo, kh, kw, stride):
    views = []
    for i in range(kh):
        for j in range(kw):
            views.append(xp[:, i:i + stride * (Ho - 1) + 1:stride,
                            j:j + stride * (Wo - 1) + 1:stride, :])
    return views


def _gconv3x3(x, w2, b2, *, stride):
    B, H, W, C = x.shape
    G = C // 128
    xp = jnp.pad(x, ((0, 0), (1, 1), (1, 1), (0, 0)))
    Ho = (H + 2 - 3) // stride + 1
    Wo = (W + 2 - 3) // stride + 1
    views = _tap_views(xp, Ho, Wo, 3, 3, stride)
    M = B * Ho * Wo
    a = jnp.stack(views, axis=0).reshape(9, M, C)
    wt = w2.reshape(9, G, 128, G, 128)
    gi = jnp.arange(G)
    wk = wt[:, gi, :, gi, :].reshape(G, 9 * 128, 128)
    bias = b2.reshape(G, 1, 128)
    tm = _pick_tm(M)
    Mp = _round_up(M, tm)
    if Mp != M:
        a = jnp.pad(a, ((0, 0), (0, Mp - M), (0, 0)))
    out = pl.pallas_call(
        _gconv_kernel,
        out_shape=jax.ShapeDtypeStruct((Mp, C), jnp.bfloat16),
        grid=(Mp // tm, G),
        in_specs=[
            pl.BlockSpec((9, tm, 128), lambda i, j: (0, i, j)),
            pl.BlockSpec((1, 9 * 128, 128), lambda i, j: (j, 0, 0)),
            pl.BlockSpec((1, 1, 128), lambda i, j: (j, 0, 0)),
        ],
        out_specs=pl.BlockSpec((tm, 128), lambda i, j: (i, j)),
        compiler_params=_cparams("parallel", "parallel"),
    )(a, wk, bias)
    if Mp != M:
        out = out[:M]
    return out.reshape(B, Ho, Wo, C)


def _stem(x_nchw, w, b):
    x = jnp.transpose(x_nchw, (0, 2, 3, 1)).astype(jnp.bfloat16)
    B, H, W, C = x.shape
    xp = jnp.pad(x, ((0, 0), (3, 3), (3, 3), (0, 0)))
    Ho = (H + 6 - 7) // 2 + 1
    Wo = (W + 6 - 7) // 2 + 1
    views = _tap_views(xp, Ho, Wo, 7, 7, 2)
    M = B * Ho * Wo
    a = jnp.stack(views, axis=3).reshape(M, 49 * C)
    a = jnp.pad(a, ((0, 0), (0, w.shape[0] - 49 * C)))
    out = _matmul(a, w, b, relu=True)
    return out.reshape(B, Ho, Wo, w.shape[1])


def _maxpool3x3s2(x):
    B, H, W, C = x.shape
    xp = jnp.pad(x, ((0, 0), (1, 1), (1, 1), (0, 0)),
                 constant_values=-jnp.inf)
    Ho = (H + 2 - 3) // 2 + 1
    Wo = (W + 2 - 3) // 2 + 1
    views = _tap_views(xp, Ho, Wo, 3, 3, 2)
    M = B * Ho * Wo
    a = jnp.stack(views, axis=0).reshape(9, M, C)
    tm = _pick_tm(M)
    out = pl.pallas_call(
        _max_kernel,
        out_shape=jax.ShapeDtypeStruct((M, C), x.dtype),
        grid=(M // tm,),
        in_specs=[pl.BlockSpec((9, tm, C), lambda i: (0, i, 0))],
        out_specs=pl.BlockSpec((tm, C), lambda i: (i, 0)),
        compiler_params=_cparams("parallel"),
    )(a)
    return out.reshape(B, Ho, Wo, C)


def _avgpool(x):
    B, H, W, C = x.shape
    a = x.reshape(B, H * W, C)
    tc = 256
    out = pl.pallas_call(
        _avg_kernel,
        out_shape=jax.ShapeDtypeStruct((B, C), jnp.float32),
        grid=(C // tc,),
        in_specs=[pl.BlockSpec((B, H * W, tc), lambda j: (0, 0, j))],
        out_specs=pl.BlockSpec((B, tc), lambda j: (0, j)),
        compiler_params=_cparams("parallel"),
    )(a)
    return out


def _bottleneck(x, w1, b1, w2, b2, w3, b3, wd, bd, stride):
    out = _conv1x1(x, w1, b1, relu=True)
    out = _gconv3x3(out, w2, b2, stride=stride)
    if wd is not None:
        identity = _conv1x1(x, wd, bd, relu=False, stride=stride)
    else:
        identity = x
    B, H, W, C = out.shape
    res = identity.reshape(B * H * W, identity.shape[-1])
    out = _matmul(out.reshape(B * H * W, C), w3, b3, residual=res, relu=True)
    return out.reshape(B, H, W, w3.shape[1])


def kernel(stem_w, stem_b, s0_b0_w1, s0_b0_b1, s0_b0_w2, s0_b0_b2, s0_b0_w3, s0_b0_b3, s0_b0_wd, s0_b0_bd, s0_b1_w1, s0_b1_b1, s0_b1_w2, s0_b1_b2, s0_b1_w3, s0_b1_b3, s0_b2_w1, s0_b2_b1, s0_b2_w2, s0_b2_b2, s0_b2_w3, s0_b2_b3, s1_b0_w1, s1_b0_b1, s1_b0_w2, s1_b0_b2, s1_b0_w3, s1_b0_b3, s1_b0_wd, s1_b0_bd, s1_b1_w1, s1_b1_b1, s1_b1_w2, s1_b1_b2, s1_b1_w3, s1_b1_b3, s1_b2_w1, s1_b2_b1, s1_b2_w2, s1_b2_b2, s1_b2_w3, s1_b2_b3, s1_b3_w1, s1_b3_b1, s1_b3_w2, s1_b3_b2, s1_b3_w3, s1_b3_b3, s2_b0_w1, s2_b0_b1, s2_b0_w2, s2_b0_b2, s2_b0_w3, s2_b0_b3, s2_b0_wd, s2_b0_bd, s2_b1_w1, s2_b1_b1, s2_b1_w2, s2_b1_b2, s2_b1_w3, s2_b1_b3, s2_b2_w1, s2_b2_b1, s2_b2_w2, s2_b2_b2, s2_b2_w3, s2_b2_b3, s2_b3_w1, s2_b3_b1, s2_b3_w2, s2_b3_b2, s2_b3_w3, s2_b3_b3, s2_b4_w1, s2_b4_b1, s2_b4_w2, s2_b4_b2, s2_b4_w3, s2_b4_b3, s2_b5_w1, s2_b5_b1, s2_b5_w2, s2_b5_b2, s2_b5_w3, s2_b5_b3, s3_b0_w1, s3_b0_b1, s3_b0_w2, s3_b0_b2, s3_b0_w3, s3_b0_b3, s3_b0_wd, s3_b0_bd, s3_b1_w1, s3_b1_b1, s3_b1_w2, s3_b1_b2, s3_b1_w3, s3_b1_b3, s3_b2_w1, s3_b2_b1, s3_b2_w2, s3_b2_b2, s3_b2_w3, s3_b2_b3, x):
    L = dict(locals())
    out = _stem(x, stem_w, stem_b)
    out = _maxpool3x3s2(out)
    for si, cnt in enumerate(_COUNTS):
        for bi in range(cnt):
            stride = 2 if (bi == 0 and si > 0) else 1
            out = _bottleneck(
                out,
                L[f"s{si}_b{bi}_w1"], L[f"s{si}_b{bi}_b1"],
                L[f"s{si}_b{bi}_w2"], L[f"s{si}_b{bi}_b2"],
                L[f"s{si}_b{bi}_w3"], L[f"s{si}_b{bi}_b3"],
                L.get(f"s{si}_b{bi}_wd"), L.get(f"s{si}_b{bi}_bd"),
                stride,
            )
    pool = _avgpool(out)
    return pool.reshape(pool.shape[0], -1, 1, 1)


# speedup vs baseline: 1.9612x; 1.1499x over previous
import functools

import jax
import jax.numpy as jnp
from jax.experimental import pallas as pl
from jax.experimental.pallas import tpu as pltpu

_VMEM = 48 * 1024 * 1024
_COUNTS = (3, 4, 6, 3)


def _round_up(x, m):
    return (x + m - 1) // m * m


def _pick_tm(M):
    for tm in (512, 448, 784, 256, 128, 64, 32, 16, 8):
        if M % tm == 0:
            return tm
    return 256


def _cparams(*sem):
    return pltpu.CompilerParams(dimension_semantics=sem,
                                vmem_limit_bytes=_VMEM)


def _mm_kernel(a_ref, w_ref, b_ref, o_ref, *, relu):
    acc = jnp.dot(a_ref[...], w_ref[...], preferred_element_type=jnp.float32)
    acc = acc + b_ref[...]
    if relu:
        acc = jnp.maximum(acc, 0.0)
    o_ref[...] = acc.astype(o_ref.dtype)


def _mm_res_kernel(a_ref, w_ref, b_ref, r_ref, o_ref, *, relu):
    acc = jnp.dot(a_ref[...], w_ref[...], preferred_element_type=jnp.float32)
    acc = acc + b_ref[...] + r_ref[...].astype(jnp.float32)
    if relu:
        acc = jnp.maximum(acc, 0.0)
    o_ref[...] = acc.astype(o_ref.dtype)


def _gconv_kernel(a_ref, w_ref, b_ref, o_ref):
    a = jnp.concatenate([a_ref[t] for t in range(9)], axis=1)
    acc = jnp.dot(a, w_ref[0], preferred_element_type=jnp.float32)
    acc = jnp.maximum(acc + b_ref[0], 0.0)
    o_ref[...] = acc.astype(o_ref.dtype)


def _max_kernel(x_ref, o_ref):
    o_ref[...] = jnp.max(x_ref[...], axis=0)


def _fused_b_kernel(x_ref, w1_ref, b1_ref, w2_ref, b2_ref, o_ref, *, W, G):
    HW = x_ref.shape[1]
    xm = x_ref[0]
    h1 = jnp.dot(xm, w1_ref[...], preferred_element_type=jnp.float32)
    h1 = jnp.maximum(h1 + b1_ref[...], 0.0).astype(jnp.bfloat16)
    col = jax.lax.broadcasted_iota(jnp.int32, (HW, 128), 0) % W
    keep_l = col != 0
    keep_r = col != (W - 1)
    for j in range(G):
        h1j = h1[:, j * 128:(j + 1) * 128]
        taps = []
        for di in (-W, 0, W):
            for dj in (-1, 0, 1):
                sh = di + dj
                if sh > 0:
                    a = jnp.concatenate(
                        [h1j[sh:], jnp.zeros((sh, 128), jnp.bfloat16)], axis=0)
                elif sh < 0:
                    a = jnp.concatenate(
                        [jnp.zeros((-sh, 128), jnp.bfloat16), h1j[:HW + sh]],
                        axis=0)
                else:
                    a = h1j
                if dj == -1:
                    a = jnp.where(keep_l, a, jnp.bfloat16(0))
                elif dj == 1:
                    a = jnp.where(keep_r, a, jnp.bfloat16(0))
                taps.append(a)
        amat = jnp.concatenate(taps, axis=1)
        acc = jnp.dot(amat, w2_ref[j], preferred_element_type=jnp.float32)
        acc = jnp.maximum(acc + b2_ref[j], 0.0)
        o_ref[0, :, j * 128:(j + 1) * 128] = acc.astype(jnp.bfloat16)


def _avg_kernel(x_ref, o_ref):
    x = x_ref[...].astype(jnp.float32)
    o_ref[...] = jnp.sum(x, axis=1) * (1.0 / x.shape[1])


def _matmul(a, w, bias, residual=None, *, relu):
    M, K = a.shape
    N = w.shape[1]
    tm = _pick_tm(M)
    Mp = _round_up(M, tm)
    if Mp != M:
        a = jnp.pad(a, ((0, Mp - M), (0, 0)))
    tn = 512 if N % 512 == 0 else (256 if N % 256 == 0 else 128)
    bias2 = bias.reshape(1, N)
    grid = (Mp // tm, N // tn)
    in_specs = [
        pl.BlockSpec((tm, K), lambda i, j: (i, 0)),
        pl.BlockSpec((K, tn), lambda i, j: (0, j)),
        pl.BlockSpec((1, tn), lambda i, j: (0, j)),
    ]
    args = [a, w, bias2]
    if residual is not None:
        r = residual
        if Mp != M:
            r = jnp.pad(r, ((0, Mp - M), (0, 0)))
        in_specs.append(pl.BlockSpec((tm, tn), lambda i, j: (i, j)))
        args.append(r)
        body = functools.partial(_mm_res_kernel, relu=relu)
    else:
        body = functools.partial(_mm_kernel, relu=relu)
    out = pl.pallas_call(
        body,
        out_shape=jax.ShapeDtypeStruct((Mp, N), jnp.bfloat16),
        grid=grid,
        in_specs=in_specs,
        out_specs=pl.BlockSpec((tm, tn), lambda i, j: (i, j)),
        compiler_params=_cparams("parallel", "parallel"),
    )(*args)
    return out[:M] if Mp != M else out


def _conv1x1(x, w, b, *, relu, stride=1, residual=None):
    if stride > 1:
        x = x[:, ::stride, ::stride, :]
    B, H, W, C = x.shape
    out = _matmul(x.reshape(B * H * W, C), w, b, residual=residual, relu=relu)
    return out.reshape(B, H, W, w.shape[1])


def _fused_conv1_gconv(x, w1, b1, w2, b2):
    B, H, W, C = x.shape
    width = w1.shape[1]
    G = width // 128
    wt = w2.reshape(9, G, 128, G, 128)
    gi = jnp.arange(G)
    wk = wt[:, gi, :, gi, :].reshape(G, 9 * 128, 128)
    out = pl.pallas_call(
        functools.partial(_fused_b_kernel, W=W, G=G),
        out_shape=jax.ShapeDtypeStruct((B, H * W, width), jnp.bfloat16),
        grid=(B,),
        in_specs=[
            pl.BlockSpec((1, H * W, C), lambda b: (b, 0, 0)),
            pl.BlockSpec((C, width), lambda b: (0, 0)),
            pl.BlockSpec((1, width), lambda b: (0, 0)),
            pl.BlockSpec((G, 9 * 128, 128), lambda b: (0, 0, 0)),
            pl.BlockSpec((G, 1, 128), lambda b: (0, 0, 0)),
        ],
        out_specs=pl.BlockSpec((1, H * W, width), lambda b: (b, 0, 0)),
        compiler_params=_cparams("parallel"),
    )(x.reshape(B, H * W, C), w1, b1.reshape(1, width), wk,
      b2.reshape(G, 1, 128))
    return out.reshape(B, H, W, width)


def _tap_views(xp, Ho, Wo, kh, kw, stride):
    views = []
    for i in range(kh):
        for j in range(kw):
            views.append(xp[:, i:i + stride * (Ho - 1) + 1:stride,
                            j:j + stride * (Wo - 1) + 1:stride, :])
    return views


def _gconv3x3(x, w2, b2, *, stride):
    B, H, W, C = x.shape
    G = C // 128
    xp = jnp.pad(x, ((0, 0), (1, 1), (1, 1), (0, 0)))
    Ho = (H + 2 - 3) // stride + 1
    Wo = (W + 2 - 3) // stride + 1
    views = _tap_views(xp, Ho, Wo, 3, 3, stride)
    M = B * Ho * Wo
    a = jnp.stack(views, axis=0).reshape(9, M, C)
    wt = w2.reshape(9, G, 128, G, 128)
    gi = jnp.arange(G)
    wk = wt[:, gi, :, gi, :].reshape(G, 9 * 128, 128)
    bias = b2.reshape(G, 1, 128)
    tm = _pick_tm(M)
    Mp = _round_up(M, tm)
    if Mp != M:
        a = jnp.pad(a, ((0, 0), (0, Mp - M), (0, 0)))
    out = pl.pallas_call(
        _gconv_kernel,
        out_shape=jax.ShapeDtypeStruct((Mp, C), jnp.bfloat16),
        grid=(Mp // tm, G),
        in_specs=[
            pl.BlockSpec((9, tm, 128), lambda i, j: (0, i, j)),
            pl.BlockSpec((1, 9 * 128, 128), lambda i, j: (j, 0, 0)),
            pl.BlockSpec((1, 1, 128), lambda i, j: (j, 0, 0)),
        ],
        out_specs=pl.BlockSpec((tm, 128), lambda i, j: (i, j)),
        compiler_params=_cparams("parallel", "parallel"),
    )(a, wk, bias)
    if Mp != M:
        out = out[:M]
    return out.reshape(B, Ho, Wo, C)


def _stem(x_nchw, w, b):
    x = jnp.transpose(x_nchw, (0, 2, 3, 1)).astype(jnp.bfloat16)
    B, H, W, C = x.shape
    xp = jnp.pad(x, ((0, 0), (3, 3), (3, 3), (0, 0)))
    Ho = (H + 6 - 7) // 2 + 1
    Wo = (W + 6 - 7) // 2 + 1
    views = _tap_views(xp, Ho, Wo, 7, 7, 2)
    M = B * Ho * Wo
    a = jnp.stack(views, axis=3).reshape(M, 49 * C)
    a = jnp.pad(a, ((0, 0), (0, w.shape[0] - 49 * C)))
    out = _matmul(a, w, b, relu=True)
    return out.reshape(B, Ho, Wo, w.shape[1])


def _maxpool3x3s2(x):
    B, H, W, C = x.shape
    xp = jnp.pad(x, ((0, 0), (1, 1), (1, 1), (0, 0)),
                 constant_values=-jnp.inf)
    Ho = (H + 2 - 3) // 2 + 1
    Wo = (W + 2 - 3) // 2 + 1
    views = _tap_views(xp, Ho, Wo, 3, 3, 2)
    M = B * Ho * Wo
    a = jnp.stack(views, axis=0).reshape(9, M, C)
    tm = _pick_tm(M)
    out = pl.pallas_call(
        _max_kernel,
        out_shape=jax.ShapeDtypeStruct((M, C), x.dtype),
        grid=(M // tm,),
        in_specs=[pl.BlockSpec((9, tm, C), lambda i: (0, i, 0))],
        out_specs=pl.BlockSpec((tm, C), lambda i: (i, 0)),
        compiler_params=_cparams("parallel"),
    )(a)
    return out.reshape(B, Ho, Wo, C)


def _avgpool(x):
    B, H, W, C = x.shape
    a = x.reshape(B, H * W, C)
    tc = 256
    out = pl.pallas_call(
        _avg_kernel,
        out_shape=jax.ShapeDtypeStruct((B, C), jnp.float32),
        grid=(C // tc,),
        in_specs=[pl.BlockSpec((B, H * W, tc), lambda j: (0, 0, j))],
        out_specs=pl.BlockSpec((B, tc), lambda j: (0, j)),
        compiler_params=_cparams("parallel"),
    )(a)
    return out


def _bottleneck(x, w1, b1, w2, b2, w3, b3, wd, bd, stride):
    if stride == 1 and x.shape[1] >= 14:
        out = _fused_conv1_gconv(x, w1, b1, w2, b2)
    else:
        out = _conv1x1(x, w1, b1, relu=True)
        out = _gconv3x3(out, w2, b2, stride=stride)
    if wd is not None:
        identity = _conv1x1(x, wd, bd, relu=False, stride=stride)
    else:
        identity = x
    B, H, W, C = out.shape
    res = identity.reshape(B * H * W, identity.shape[-1])
    out = _matmul(out.reshape(B * H * W, C), w3, b3, residual=res, relu=True)
    return out.reshape(B, H, W, w3.shape[1])


def kernel(stem_w, stem_b, s0_b0_w1, s0_b0_b1, s0_b0_w2, s0_b0_b2, s0_b0_w3, s0_b0_b3, s0_b0_wd, s0_b0_bd, s0_b1_w1, s0_b1_b1, s0_b1_w2, s0_b1_b2, s0_b1_w3, s0_b1_b3, s0_b2_w1, s0_b2_b1, s0_b2_w2, s0_b2_b2, s0_b2_w3, s0_b2_b3, s1_b0_w1, s1_b0_b1, s1_b0_w2, s1_b0_b2, s1_b0_w3, s1_b0_b3, s1_b0_wd, s1_b0_bd, s1_b1_w1, s1_b1_b1, s1_b1_w2, s1_b1_b2, s1_b1_w3, s1_b1_b3, s1_b2_w1, s1_b2_b1, s1_b2_w2, s1_b2_b2, s1_b2_w3, s1_b2_b3, s1_b3_w1, s1_b3_b1, s1_b3_w2, s1_b3_b2, s1_b3_w3, s1_b3_b3, s2_b0_w1, s2_b0_b1, s2_b0_w2, s2_b0_b2, s2_b0_w3, s2_b0_b3, s2_b0_wd, s2_b0_bd, s2_b1_w1, s2_b1_b1, s2_b1_w2, s2_b1_b2, s2_b1_w3, s2_b1_b3, s2_b2_w1, s2_b2_b1, s2_b2_w2, s2_b2_b2, s2_b2_w3, s2_b2_b3, s2_b3_w1, s2_b3_b1, s2_b3_w2, s2_b3_b2, s2_b3_w3, s2_b3_b3, s2_b4_w1, s2_b4_b1, s2_b4_w2, s2_b4_b2, s2_b4_w3, s2_b4_b3, s2_b5_w1, s2_b5_b1, s2_b5_w2, s2_b5_b2, s2_b5_w3, s2_b5_b3, s3_b0_w1, s3_b0_b1, s3_b0_w2, s3_b0_b2, s3_b0_w3, s3_b0_b3, s3_b0_wd, s3_b0_bd, s3_b1_w1, s3_b1_b1, s3_b1_w2, s3_b1_b2, s3_b1_w3, s3_b1_b3, s3_b2_w1, s3_b2_b1, s3_b2_w2, s3_b2_b2, s3_b2_w3, s3_b2_b3, x):
    L = dict(locals())
    out = _stem(x, stem_w, stem_b)
    out = _maxpool3x3s2(out)
    for si, cnt in enumerate(_COUNTS):
        for bi in range(cnt):
            stride = 2 if (bi == 0 and si > 0) else 1
            out = _bottleneck(
                out,
                L[f"s{si}_b{bi}_w1"], L[f"s{si}_b{bi}_b1"],
                L[f"s{si}_b{bi}_w2"], L[f"s{si}_b{bi}_b2"],
                L[f"s{si}_b{bi}_w3"], L[f"s{si}_b{bi}_b3"],
                L.get(f"s{si}_b{bi}_wd"), L.get(f"s{si}_b{bi}_bd"),
                stride,
            )
    pool = _avgpool(out)
    return pool.reshape(pool.shape[0], -1, 1, 1)


# speedup vs baseline: 2.3941x; 1.2208x over previous
import functools

import jax
import jax.numpy as jnp
from jax.experimental import pallas as pl
from jax.experimental.pallas import tpu as pltpu

_VMEM = 48 * 1024 * 1024
_COUNTS = (3, 4, 6, 3)


def _round_up(x, m):
    return (x + m - 1) // m * m


def _pick_tm(M):
    for tm in (512, 448, 784, 256, 128, 64, 32, 16, 8):
        if M % tm == 0:
            return tm
    return 256


def _cparams(*sem):
    return pltpu.CompilerParams(dimension_semantics=sem,
                                vmem_limit_bytes=_VMEM)


def _mm_kernel(a_ref, w_ref, b_ref, o_ref, *, relu):
    acc = jnp.dot(a_ref[...], w_ref[...], preferred_element_type=jnp.float32)
    acc = acc + b_ref[...]
    if relu:
        acc = jnp.maximum(acc, 0.0)
    o_ref[...] = acc.astype(o_ref.dtype)


def _mm_res_kernel(a_ref, w_ref, b_ref, r_ref, o_ref, *, relu):
    acc = jnp.dot(a_ref[...], w_ref[...], preferred_element_type=jnp.float32)
    acc = acc + b_ref[...] + r_ref[...].astype(jnp.float32)
    if relu:
        acc = jnp.maximum(acc, 0.0)
    o_ref[...] = acc.astype(o_ref.dtype)


def _gconv_kernel(a_ref, w_ref, b_ref, o_ref):
    a = jnp.concatenate([a_ref[t] for t in range(9)], axis=1)
    acc = jnp.dot(a, w_ref[0], preferred_element_type=jnp.float32)
    acc = jnp.maximum(acc + b_ref[0], 0.0)
    o_ref[...] = acc.astype(o_ref.dtype)


def _max_kernel(x_ref, o_ref):
    o_ref[...] = jnp.max(x_ref[...], axis=0)


def _fused_b_kernel(x_ref, w1_ref, b1_ref, w2_ref, b2_ref, o_ref, *, W, G):
    HW = x_ref.shape[1]
    xm = x_ref[0]
    h1 = jnp.dot(xm, w1_ref[...], preferred_element_type=jnp.float32)
    h1 = jnp.maximum(h1 + b1_ref[...], 0.0).astype(jnp.bfloat16)
    col = jax.lax.broadcasted_iota(jnp.int32, (HW, 128), 0) % W
    keep_l = col != 0
    keep_r = col != (W - 1)
    for j in range(G):
        h1j = h1[:, j * 128:(j + 1) * 128]
        taps = []
        for di in (-W, 0, W):
            for dj in (-1, 0, 1):
                sh = di + dj
                if sh > 0:
                    a = jnp.concatenate(
                        [h1j[sh:], jnp.zeros((sh, 128), jnp.bfloat16)], axis=0)
                elif sh < 0:
                    a = jnp.concatenate(
                        [jnp.zeros((-sh, 128), jnp.bfloat16), h1j[:HW + sh]],
                        axis=0)
                else:
                    a = h1j
                if dj == -1:
                    a = jnp.where(keep_l, a, jnp.bfloat16(0))
                elif dj == 1:
                    a = jnp.where(keep_r, a, jnp.bfloat16(0))
                taps.append(a)
        amat = jnp.concatenate(taps, axis=1)
        acc = jnp.dot(amat, w2_ref[j], preferred_element_type=jnp.float32)
        acc = jnp.maximum(acc + b2_ref[j], 0.0)
        o_ref[0, :, j * 128:(j + 1) * 128] = acc.astype(jnp.bfloat16)


def _avg_kernel(x_ref, o_ref):
    x = x_ref[...].astype(jnp.float32)
    o_ref[...] = jnp.sum(x, axis=1) * (1.0 / x.shape[1])


def _matmul(a, w, bias, residual=None, *, relu):
    M, K = a.shape
    N = w.shape[1]
    tm = _pick_tm(M)
    Mp = _round_up(M, tm)
    if Mp != M:
        a = jnp.pad(a, ((0, Mp - M), (0, 0)))
    tn = 512 if N % 512 == 0 else (256 if N % 256 == 0 else 128)
    bias2 = bias.reshape(1, N)
    grid = (Mp // tm, N // tn)
    in_specs = [
        pl.BlockSpec((tm, K), lambda i, j: (i, 0)),
        pl.BlockSpec((K, tn), lambda i, j: (0, j)),
        pl.BlockSpec((1, tn), lambda i, j: (0, j)),
    ]
    args = [a, w, bias2]
    if residual is not None:
        r = residual
        if Mp != M:
            r = jnp.pad(r, ((0, Mp - M), (0, 0)))
        in_specs.append(pl.BlockSpec((tm, tn), lambda i, j: (i, j)))
        args.append(r)
        body = functools.partial(_mm_res_kernel, relu=relu)
    else:
        body = functools.partial(_mm_kernel, relu=relu)
    out = pl.pallas_call(
        body,
        out_shape=jax.ShapeDtypeStruct((Mp, N), jnp.bfloat16),
        grid=grid,
        in_specs=in_specs,
        out_specs=pl.BlockSpec((tm, tn), lambda i, j: (i, j)),
        compiler_params=_cparams("parallel", "parallel"),
    )(*args)
    return out[:M] if Mp != M else out


def _conv1x1(x, w, b, *, relu, stride=1, residual=None):
    if stride > 1:
        x = x[:, ::stride, ::stride, :]
    B, H, W, C = x.shape
    out = _matmul(x.reshape(B * H * W, C), w, b, residual=residual, relu=relu)
    return out.reshape(B, H, W, w.shape[1])


def _fused_conv1_gconv(x, w1, b1, w2, b2):
    B, H, W, C = x.shape
    width = w1.shape[1]
    G = width // 128
    wt = w2.reshape(9, G, 128, G, 128)
    gi = jnp.arange(G)
    wk = wt[:, gi, :, gi, :].reshape(G, 9 * 128, 128)
    out = pl.pallas_call(
        functools.partial(_fused_b_kernel, W=W, G=G),
        out_shape=jax.ShapeDtypeStruct((B, H * W, width), jnp.bfloat16),
        grid=(B,),
        in_specs=[
            pl.BlockSpec((1, H * W, C), lambda b: (b, 0, 0)),
            pl.BlockSpec((C, width), lambda b: (0, 0)),
            pl.BlockSpec((1, width), lambda b: (0, 0)),
            pl.BlockSpec((G, 9 * 128, 128), lambda b: (0, 0, 0)),
            pl.BlockSpec((G, 1, 128), lambda b: (0, 0, 0)),
        ],
        out_specs=pl.BlockSpec((1, H * W, width), lambda b: (b, 0, 0)),
        compiler_params=_cparams("parallel"),
    )(x.reshape(B, H * W, C), w1, b1.reshape(1, width), wk,
      b2.reshape(G, 1, 128))
    return out.reshape(B, H, W, width)


def _tap_views(xp, Ho, Wo, kh, kw, stride):
    views = []
    for i in range(kh):
        for j in range(kw):
            views.append(xp[:, i:i + stride * (Ho - 1) + 1:stride,
                            j:j + stride * (Wo - 1) + 1:stride, :])
    return views


def _gconv3x3(x, w2, b2, *, stride):
    B, H, W, C = x.shape
    G = C // 128
    xp = jnp.pad(x, ((0, 0), (1, 1), (1, 1), (0, 0)))
    Ho = (H + 2 - 3) // stride + 1
    Wo = (W + 2 - 3) // stride + 1
    views = _tap_views(xp, Ho, Wo, 3, 3, stride)
    M = B * Ho * Wo
    a = jnp.stack(views, axis=0).reshape(9, M, C)
    wt = w2.reshape(9, G, 128, G, 128)
    gi = jnp.arange(G)
    wk = wt[:, gi, :, gi, :].reshape(G, 9 * 128, 128)
    bias = b2.reshape(G, 1, 128)
    tm = _pick_tm(M)
    Mp = _round_up(M, tm)
    if Mp != M:
        a = jnp.pad(a, ((0, 0), (0, Mp - M), (0, 0)))
    out = pl.pallas_call(
        _gconv_kernel,
        out_shape=jax.ShapeDtypeStruct((Mp, C), jnp.bfloat16),
        grid=(Mp // tm, G),
        in_specs=[
            pl.BlockSpec((9, tm, 128), lambda i, j: (0, i, j)),
            pl.BlockSpec((1, 9 * 128, 128), lambda i, j: (j, 0, 0)),
            pl.BlockSpec((1, 1, 128), lambda i, j: (j, 0, 0)),
        ],
        out_specs=pl.BlockSpec((tm, 128), lambda i, j: (i, j)),
        compiler_params=_cparams("parallel", "parallel"),
    )(a, wk, bias)
    if Mp != M:
        out = out[:M]
    return out.reshape(B, Ho, Wo, C)


def _stem(x_nchw, w, b):
    x = jnp.transpose(x_nchw, (0, 2, 3, 1)).astype(jnp.bfloat16)
    B, H, W, C = x.shape
    Ho, Wo = H // 2, W // 2
    xs = jnp.pad(x, ((0, 0), (3, 3), (3, 3), (0, 0)))
    Y = xs.reshape(B, Ho + 3, 2, Wo + 3, 2, C)
    Y = jnp.transpose(Y, (0, 1, 3, 2, 4, 5)).reshape(B, Ho + 3, Wo + 3, 4 * C)
    views = []
    for a_ in range(4):
        for b_ in range(4):
            views.append(Y[:, a_:a_ + Ho, b_:b_ + Wo, :])
    M = B * Ho * Wo
    amat = jnp.stack(views, axis=3).reshape(M, 16 * 4 * C)
    Kp = w.shape[0]
    amat = jnp.pad(amat, ((0, 0), (0, Kp - 16 * 4 * C)))
    idx = []
    for a_ in range(4):
        for b_ in range(4):
            for p in range(2):
                for q in range(2):
                    for c in range(C):
                        i, j = 2 * a_ + p, 2 * b_ + q
                        idx.append(((i * 7 + j) * C + c) if i < 7 and j < 7
                                   else Kp - 1)
    idx += [Kp - 1] * (Kp - len(idx))
    w4 = jnp.take(w, jnp.array(idx), axis=0)
    out = _matmul(amat, w4, b, relu=True)
    return out.reshape(B, Ho, Wo, w.shape[1])


def _maxpool3x3s2(x):
    B, H, W, C = x.shape
    xp = jnp.pad(x, ((0, 0), (1, 1), (1, 1), (0, 0)),
                 constant_values=-jnp.inf)
    Ho = (H + 2 - 3) // 2 + 1
    Wo = (W + 2 - 3) // 2 + 1
    views = _tap_views(xp, Ho, Wo, 3, 3, 2)
    M = B * Ho * Wo
    a = jnp.stack(views, axis=0).reshape(9, M, C)
    tm = _pick_tm(M)
    out = pl.pallas_call(
        _max_kernel,
        out_shape=jax.ShapeDtypeStruct((M, C), x.dtype),
        grid=(M // tm,),
        in_specs=[pl.BlockSpec((9, tm, C), lambda i: (0, i, 0))],
        out_specs=pl.BlockSpec((tm, C), lambda i: (i, 0)),
        compiler_params=_cparams("parallel"),
    )(a)
    return out.reshape(B, Ho, Wo, C)


def _avgpool(x):
    B, H, W, C = x.shape
    a = x.reshape(B, H * W, C)
    tc = 256
    out = pl.pallas_call(
        _avg_kernel,
        out_shape=jax.ShapeDtypeStruct((B, C), jnp.float32),
        grid=(C // tc,),
        in_specs=[pl.BlockSpec((B, H * W, tc), lambda j: (0, 0, j))],
        out_specs=pl.BlockSpec((B, tc), lambda j: (0, j)),
        compiler_params=_cparams("parallel"),
    )(a)
    return out


def _bottleneck(x, w1, b1, w2, b2, w3, b3, wd, bd, stride):
    if stride == 1 and x.shape[1] >= 14:
        out = _fused_conv1_gconv(x, w1, b1, w2, b2)
    else:
        out = _conv1x1(x, w1, b1, relu=True)
        out = _gconv3x3(out, w2, b2, stride=stride)
    if wd is not None:
        identity = _conv1x1(x, wd, bd, relu=False, stride=stride)
    else:
        identity = x
    B, H, W, C = out.shape
    res = identity.reshape(B * H * W, identity.shape[-1])
    out = _matmul(out.reshape(B * H * W, C), w3, b3, residual=res, relu=True)
    return out.reshape(B, H, W, w3.shape[1])


def kernel(stem_w, stem_b, s0_b0_w1, s0_b0_b1, s0_b0_w2, s0_b0_b2, s0_b0_w3, s0_b0_b3, s0_b0_wd, s0_b0_bd, s0_b1_w1, s0_b1_b1, s0_b1_w2, s0_b1_b2, s0_b1_w3, s0_b1_b3, s0_b2_w1, s0_b2_b1, s0_b2_w2, s0_b2_b2, s0_b2_w3, s0_b2_b3, s1_b0_w1, s1_b0_b1, s1_b0_w2, s1_b0_b2, s1_b0_w3, s1_b0_b3, s1_b0_wd, s1_b0_bd, s1_b1_w1, s1_b1_b1, s1_b1_w2, s1_b1_b2, s1_b1_w3, s1_b1_b3, s1_b2_w1, s1_b2_b1, s1_b2_w2, s1_b2_b2, s1_b2_w3, s1_b2_b3, s1_b3_w1, s1_b3_b1, s1_b3_w2, s1_b3_b2, s1_b3_w3, s1_b3_b3, s2_b0_w1, s2_b0_b1, s2_b0_w2, s2_b0_b2, s2_b0_w3, s2_b0_b3, s2_b0_wd, s2_b0_bd, s2_b1_w1, s2_b1_b1, s2_b1_w2, s2_b1_b2, s2_b1_w3, s2_b1_b3, s2_b2_w1, s2_b2_b1, s2_b2_w2, s2_b2_b2, s2_b2_w3, s2_b2_b3, s2_b3_w1, s2_b3_b1, s2_b3_w2, s2_b3_b2, s2_b3_w3, s2_b3_b3, s2_b4_w1, s2_b4_b1, s2_b4_w2, s2_b4_b2, s2_b4_w3, s2_b4_b3, s2_b5_w1, s2_b5_b1, s2_b5_w2, s2_b5_b2, s2_b5_w3, s2_b5_b3, s3_b0_w1, s3_b0_b1, s3_b0_w2, s3_b0_b2, s3_b0_w3, s3_b0_b3, s3_b0_wd, s3_b0_bd, s3_b1_w1, s3_b1_b1, s3_b1_w2, s3_b1_b2, s3_b1_w3, s3_b1_b3, s3_b2_w1, s3_b2_b1, s3_b2_w2, s3_b2_b2, s3_b2_w3, s3_b2_b3, x):
    L = dict(locals())
    out = _stem(x, stem_w, stem_b)
    out = _maxpool3x3s2(out)
    for si, cnt in enumerate(_COUNTS):
        for bi in range(cnt):
            stride = 2 if (bi == 0 and si > 0) else 1
            out = _bottleneck(
                out,
                L[f"s{si}_b{bi}_w1"], L[f"s{si}_b{bi}_b1"],
                L[f"s{si}_b{bi}_w2"], L[f"s{si}_b{bi}_b2"],
                L[f"s{si}_b{bi}_w3"], L[f"s{si}_b{bi}_b3"],
                L.get(f"s{si}_b{bi}_wd"), L.get(f"s{si}_b{bi}_bd"),
                stride,
            )
    pool = _avgpool(out)
    return pool.reshape(pool.shape[0], -1, 1, 1)


# speedup vs baseline: 3.6947x; 1.5433x over previous
import functools

import jax
import jax.numpy as jnp
from jax.experimental import pallas as pl
from jax.experimental.pallas import tpu as pltpu

_VMEM = 48 * 1024 * 1024
_COUNTS = (3, 4, 6, 3)


def _round_up(x, m):
    return (x + m - 1) // m * m


def _pick_tm(M):
    for tm in (512, 448, 784, 256, 128, 64, 32, 16, 8):
        if M % tm == 0:
            return tm
    return 256


def _cparams(*sem):
    return pltpu.CompilerParams(dimension_semantics=sem,
                                vmem_limit_bytes=_VMEM)


def _mm_kernel(a_ref, w_ref, b_ref, o_ref, *, relu):
    acc = jnp.dot(a_ref[...], w_ref[...], preferred_element_type=jnp.float32)
    acc = acc + b_ref[...]
    if relu:
        acc = jnp.maximum(acc, 0.0)
    o_ref[...] = acc.astype(o_ref.dtype)


def _mm_res_kernel(a_ref, w_ref, b_ref, r_ref, o_ref, *, relu):
    acc = jnp.dot(a_ref[...], w_ref[...], preferred_element_type=jnp.float32)
    acc = acc + b_ref[...] + r_ref[...].astype(jnp.float32)
    if relu:
        acc = jnp.maximum(acc, 0.0)
    o_ref[...] = acc.astype(o_ref.dtype)


def _gconv_kernel(a_ref, w_ref, b_ref, o_ref):
    a = jnp.concatenate([a_ref[t] for t in range(9)], axis=1)
    acc = jnp.dot(a, w_ref[0], preferred_element_type=jnp.float32)
    acc = jnp.maximum(acc + b_ref[0], 0.0)
    o_ref[...] = acc.astype(o_ref.dtype)


def _max_kernel(x_ref, o_ref):
    o_ref[...] = jnp.max(x_ref[...], axis=0)


def _fused_b_kernel(x_ref, w1_ref, b1_ref, w2_ref, b2_ref, o_ref, *, W, G):
    HW = x_ref.shape[1]
    xm = x_ref[0]
    h1 = jnp.dot(xm, w1_ref[...], preferred_element_type=jnp.float32)
    h1 = jnp.maximum(h1 + b1_ref[...], 0.0).astype(jnp.bfloat16)
    col = jax.lax.broadcasted_iota(jnp.int32, (HW, 128), 0) % W
    keep_l = col != 0
    keep_r = col != (W - 1)
    for j in range(G):
        h1j = h1[:, j * 128:(j + 1) * 128]
        taps = []
        for di in (-W, 0, W):
            for dj in (-1, 0, 1):
                sh = di + dj
                if sh > 0:
                    a = jnp.concatenate(
                        [h1j[sh:], jnp.zeros((sh, 128), jnp.bfloat16)], axis=0)
                elif sh < 0:
                    a = jnp.concatenate(
                        [jnp.zeros((-sh, 128), jnp.bfloat16), h1j[:HW + sh]],
                        axis=0)
                else:
                    a = h1j
                if dj == -1:
                    a = jnp.where(keep_l, a, jnp.bfloat16(0))
                elif dj == 1:
                    a = jnp.where(keep_r, a, jnp.bfloat16(0))
                taps.append(a)
        amat = jnp.concatenate(taps, axis=1)
        acc = jnp.dot(amat, w2_ref[j], preferred_element_type=jnp.float32)
        acc = jnp.maximum(acc + b2_ref[j], 0.0)
        o_ref[0, :, j * 128:(j + 1) * 128] = acc.astype(jnp.bfloat16)


def _avg_kernel(x_ref, o_ref):
    x = x_ref[...].astype(jnp.float32)
    o_ref[...] = jnp.sum(x, axis=1) * (1.0 / x.shape[1])


def _matmul(a, w, bias, residual=None, *, relu):
    M, K = a.shape
    N = w.shape[1]
    tm = _pick_tm(M)
    Mp = _round_up(M, tm)
    if Mp != M:
        a = jnp.pad(a, ((0, Mp - M), (0, 0)))
    tn = 512 if N % 512 == 0 else (256 if N % 256 == 0 else 128)
    bias2 = bias.reshape(1, N)
    grid = (Mp // tm, N // tn)
    in_specs = [
        pl.BlockSpec((tm, K), lambda i, j: (i, 0)),
        pl.BlockSpec((K, tn), lambda i, j: (0, j)),
        pl.BlockSpec((1, tn), lambda i, j: (0, j)),
    ]
    args = [a, w, bias2]
    if residual is not None:
        r = residual
        if Mp != M:
            r = jnp.pad(r, ((0, Mp - M), (0, 0)))
        in_specs.append(pl.BlockSpec((tm, tn), lambda i, j: (i, j)))
        args.append(r)
        body = functools.partial(_mm_res_kernel, relu=relu)
    else:
        body = functools.partial(_mm_kernel, relu=relu)
    out = pl.pallas_call(
        body,
        out_shape=jax.ShapeDtypeStruct((Mp, N), jnp.bfloat16),
        grid=grid,
        in_specs=in_specs,
        out_specs=pl.BlockSpec((tm, tn), lambda i, j: (i, j)),
        compiler_params=_cparams("parallel", "parallel"),
    )(*args)
    return out[:M] if Mp != M else out


def _conv1x1(x, w, b, *, relu, stride=1, residual=None):
    if stride > 1:
        x = x[:, ::stride, ::stride, :]
    B, H, W, C = x.shape
    out = _matmul(x.reshape(B * H * W, C), w, b, residual=residual, relu=relu)
    return out.reshape(B, H, W, w.shape[1])


def _fused_conv1_gconv(x, w1, b1, w2, b2):
    B, H, W, C = x.shape
    width = w1.shape[1]
    G = width // 128
    wt = w2.reshape(9, G, 128, G, 128)
    gi = jnp.arange(G)
    wk = wt[:, gi, :, gi, :].reshape(G, 9 * 128, 128)
    out = pl.pallas_call(
        functools.partial(_fused_b_kernel, W=W, G=G),
        out_shape=jax.ShapeDtypeStruct((B, H * W, width), jnp.bfloat16),
        grid=(B,),
        in_specs=[
            pl.BlockSpec((1, H * W, C), lambda b: (b, 0, 0)),
            pl.BlockSpec((C, width), lambda b: (0, 0)),
            pl.BlockSpec((1, width), lambda b: (0, 0)),
            pl.BlockSpec((G, 9 * 128, 128), lambda b: (0, 0, 0)),
            pl.BlockSpec((G, 1, 128), lambda b: (0, 0, 0)),
        ],
        out_specs=pl.BlockSpec((1, H * W, width), lambda b: (b, 0, 0)),
        compiler_params=_cparams("parallel"),
    )(x.reshape(B, H * W, C), w1, b1.reshape(1, width), wk,
      b2.reshape(G, 1, 128))
    return out.reshape(B, H, W, width)


def _tap_views(xp, Ho, Wo, kh, kw, stride):
    views = []
    for i in range(kh):
        for j in range(kw):
            views.append(xp[:, i:i + stride * (Ho - 1) + 1:stride,
                            j:j + stride * (Wo - 1) + 1:stride, :])
    return views


def _gconv3x3(x, w2, b2, *, stride):
    B, H, W, C = x.shape
    G = C // 128
    xp = jnp.pad(x, ((0, 0), (1, 1), (1, 1), (0, 0)))
    Ho = (H + 2 - 3) // stride + 1
    Wo = (W + 2 - 3) // stride + 1
    views = _tap_views(xp, Ho, Wo, 3, 3, stride)
    M = B * Ho * Wo
    a = jnp.stack(views, axis=0).reshape(9, M, C)
    wt = w2.reshape(9, G, 128, G, 128)
    gi = jnp.arange(G)
    wk = wt[:, gi, :, gi, :].reshape(G, 9 * 128, 128)
    bias = b2.reshape(G, 1, 128)
    tm = _pick_tm(M)
    Mp = _round_up(M, tm)
    if Mp != M:
        a = jnp.pad(a, ((0, 0), (0, Mp - M), (0, 0)))
    out = pl.pallas_call(
        _gconv_kernel,
        out_shape=jax.ShapeDtypeStruct((Mp, C), jnp.bfloat16),
        grid=(Mp // tm, G),
        in_specs=[
            pl.BlockSpec((9, tm, 128), lambda i, j: (0, i, j)),
            pl.BlockSpec((1, 9 * 128, 128), lambda i, j: (j, 0, 0)),
            pl.BlockSpec((1, 1, 128), lambda i, j: (j, 0, 0)),
        ],
        out_specs=pl.BlockSpec((tm, 128), lambda i, j: (i, j)),
        compiler_params=_cparams("parallel", "parallel"),
    )(a, wk, bias)
    if Mp != M:
        out = out[:M]
    return out.reshape(B, Ho, Wo, C)


def _stem(x_nchw, w, b):
    x = jnp.transpose(x_nchw, (0, 2, 3, 1)).astype(jnp.bfloat16)
    B, H, W, C = x.shape
    Ho, Wo = H // 2, W // 2
    xs = jnp.pad(x, ((0, 0), (3, 3), (3, 3), (0, 0)))
    Y = xs.reshape(B, Ho + 3, 2, Wo + 3, 2, C)
    Y = jnp.transpose(Y, (0, 1, 3, 2, 4, 5)).reshape(B, Ho + 3, Wo + 3, 4 * C)
    views = []
    for a_ in range(4):
        for b_ in range(4):
            views.append(Y[:, a_:a_ + Ho, b_:b_ + Wo, :])
    M = B * Ho * Wo
    amat = jnp.stack(views, axis=3).reshape(M, 16 * 4 * C)
    Kp = w.shape[0]
    amat = jnp.pad(amat, ((0, 0), (0, Kp - 16 * 4 * C)))
    idx = []
    for a_ in range(4):
        for b_ in range(4):
            for p in range(2):
                for q in range(2):
                    for c in range(C):
                        i, j = 2 * a_ + p, 2 * b_ + q
                        idx.append(((i * 7 + j) * C + c) if i < 7 and j < 7
                                   else Kp - 1)
    idx += [Kp - 1] * (Kp - len(idx))
    w4 = jnp.take(w, jnp.array(idx), axis=0)
    out = _matmul(amat, w4, b, relu=True)
    return out.reshape(B, Ho, Wo, w.shape[1])


def _maxpool3x3s2(x):
    B, H, W, C = x.shape
    Ho, Wo = H // 2, W // 2
    z = x.reshape(B, Ho, 2, Wo, 2, C)
    zz = jnp.transpose(z, (0, 2, 4, 1, 3, 5))
    ninf = -jnp.inf
    views = []
    for p, rs in ((1, 1), (0, 0), (1, 0)):
        for q, cs in ((1, 1), (0, 0), (1, 0)):
            v = zz[:, p, q]
            if rs:
                v = jnp.pad(v, ((0, 0), (1, 0), (0, 0), (0, 0)),
                            constant_values=ninf)[:, :Ho]
            if cs:
                v = jnp.pad(v, ((0, 0), (0, 0), (1, 0), (0, 0)),
                            constant_values=ninf)[:, :, :Wo]
            views.append(v)
    M = B * Ho * Wo
    a = jnp.stack(views, axis=0).reshape(9, M, C)
    tm = _pick_tm(M)
    out = pl.pallas_call(
        _max_kernel,
        out_shape=jax.ShapeDtypeStruct((M, C), x.dtype),
        grid=(M // tm,),
        in_specs=[pl.BlockSpec((9, tm, C), lambda i: (0, i, 0))],
        out_specs=pl.BlockSpec((tm, C), lambda i: (i, 0)),
        compiler_params=_cparams("parallel"),
    )(a)
    return out.reshape(B, Ho, Wo, C)


def _avgpool(x):
    B, H, W, C = x.shape
    a = x.reshape(B, H * W, C)
    tc = 256
    out = pl.pallas_call(
        _avg_kernel,
        out_shape=jax.ShapeDtypeStruct((B, C), jnp.float32),
        grid=(C // tc,),
        in_specs=[pl.BlockSpec((B, H * W, tc), lambda j: (0, 0, j))],
        out_specs=pl.BlockSpec((B, tc), lambda j: (0, j)),
        compiler_params=_cparams("parallel"),
    )(a)
    return out


def _bottleneck(x, w1, b1, w2, b2, w3, b3, wd, bd, stride):
    if stride == 1 and x.shape[1] >= 14:
        out = _fused_conv1_gconv(x, w1, b1, w2, b2)
    else:
        out = _conv1x1(x, w1, b1, relu=True)
        out = _gconv3x3(out, w2, b2, stride=stride)
    if wd is not None:
        identity = _conv1x1(x, wd, bd, relu=False, stride=stride)
    else:
        identity = x
    B, H, W, C = out.shape
    res = identity.reshape(B * H * W, identity.shape[-1])
    out = _matmul(out.reshape(B * H * W, C), w3, b3, residual=res, relu=True)
    return out.reshape(B, H, W, w3.shape[1])


def kernel(stem_w, stem_b, s0_b0_w1, s0_b0_b1, s0_b0_w2, s0_b0_b2, s0_b0_w3, s0_b0_b3, s0_b0_wd, s0_b0_bd, s0_b1_w1, s0_b1_b1, s0_b1_w2, s0_b1_b2, s0_b1_w3, s0_b1_b3, s0_b2_w1, s0_b2_b1, s0_b2_w2, s0_b2_b2, s0_b2_w3, s0_b2_b3, s1_b0_w1, s1_b0_b1, s1_b0_w2, s1_b0_b2, s1_b0_w3, s1_b0_b3, s1_b0_wd, s1_b0_bd, s1_b1_w1, s1_b1_b1, s1_b1_w2, s1_b1_b2, s1_b1_w3, s1_b1_b3, s1_b2_w1, s1_b2_b1, s1_b2_w2, s1_b2_b2, s1_b2_w3, s1_b2_b3, s1_b3_w1, s1_b3_b1, s1_b3_w2, s1_b3_b2, s1_b3_w3, s1_b3_b3, s2_b0_w1, s2_b0_b1, s2_b0_w2, s2_b0_b2, s2_b0_w3, s2_b0_b3, s2_b0_wd, s2_b0_bd, s2_b1_w1, s2_b1_b1, s2_b1_w2, s2_b1_b2, s2_b1_w3, s2_b1_b3, s2_b2_w1, s2_b2_b1, s2_b2_w2, s2_b2_b2, s2_b2_w3, s2_b2_b3, s2_b3_w1, s2_b3_b1, s2_b3_w2, s2_b3_b2, s2_b3_w3, s2_b3_b3, s2_b4_w1, s2_b4_b1, s2_b4_w2, s2_b4_b2, s2_b4_w3, s2_b4_b3, s2_b5_w1, s2_b5_b1, s2_b5_w2, s2_b5_b2, s2_b5_w3, s2_b5_b3, s3_b0_w1, s3_b0_b1, s3_b0_w2, s3_b0_b2, s3_b0_w3, s3_b0_b3, s3_b0_wd, s3_b0_bd, s3_b1_w1, s3_b1_b1, s3_b1_w2, s3_b1_b2, s3_b1_w3, s3_b1_b3, s3_b2_w1, s3_b2_b1, s3_b2_w2, s3_b2_b2, s3_b2_w3, s3_b2_b3, x):
    L = dict(locals())
    out = _stem(x, stem_w, stem_b)
    out = _maxpool3x3s2(out)
    for si, cnt in enumerate(_COUNTS):
        for bi in range(cnt):
            stride = 2 if (bi == 0 and si > 0) else 1
            out = _bottleneck(
                out,
                L[f"s{si}_b{bi}_w1"], L[f"s{si}_b{bi}_b1"],
                L[f"s{si}_b{bi}_w2"], L[f"s{si}_b{bi}_b2"],
                L[f"s{si}_b{bi}_w3"], L[f"s{si}_b{bi}_b3"],
                L.get(f"s{si}_b{bi}_wd"), L.get(f"s{si}_b{bi}_bd"),
                stride,
            )
    pool = _avgpool(out)
    return pool.reshape(pool.shape[0], -1, 1, 1)


# speedup vs baseline: 7.5084x; 2.0322x over previous
import functools

import jax
import jax.numpy as jnp
from jax.experimental import pallas as pl
from jax.experimental.pallas import tpu as pltpu

_VMEM = 48 * 1024 * 1024
_COUNTS = (3, 4, 6, 3)


def _round_up(x, m):
    return (x + m - 1) // m * m


def _pick_tm(M):
    for tm in (512, 448, 784, 256, 128, 64, 32, 16, 8):
        if M % tm == 0:
            return tm
    return 256


def _cparams(*sem):
    return pltpu.CompilerParams(dimension_semantics=sem,
                                vmem_limit_bytes=_VMEM)


def _mm_kernel(a_ref, w_ref, b_ref, o_ref, *, relu):
    acc = jnp.dot(a_ref[...], w_ref[...], preferred_element_type=jnp.float32)
    acc = acc + b_ref[...]
    if relu:
        acc = jnp.maximum(acc, 0.0)
    o_ref[...] = acc.astype(o_ref.dtype)


def _mm_res_kernel(a_ref, w_ref, b_ref, r_ref, o_ref, *, relu):
    acc = jnp.dot(a_ref[...], w_ref[...], preferred_element_type=jnp.float32)
    acc = acc + b_ref[...] + r_ref[...].astype(jnp.float32)
    if relu:
        acc = jnp.maximum(acc, 0.0)
    o_ref[...] = acc.astype(o_ref.dtype)


def _gconv_kernel(a_ref, w_ref, b_ref, o_ref):
    a = jnp.concatenate([a_ref[t] for t in range(9)], axis=1)
    acc = jnp.dot(a, w_ref[0], preferred_element_type=jnp.float32)
    acc = jnp.maximum(acc + b_ref[0], 0.0)
    o_ref[...] = acc.astype(o_ref.dtype)


def _max_kernel(x_ref, o_ref):
    o_ref[...] = jnp.max(x_ref[...], axis=0)


def _fused_b_kernel(x_ref, w1_ref, b1_ref, w2_ref, b2_ref, o_ref, *, W, G):
    HW = x_ref.shape[1]
    xm = x_ref[0]
    h1 = jnp.dot(xm, w1_ref[...], preferred_element_type=jnp.float32)
    h1 = jnp.maximum(h1 + b1_ref[...], 0.0).astype(jnp.bfloat16)
    col = jax.lax.broadcasted_iota(jnp.int32, (HW, 128), 0) % W
    keep_l = col != 0
    keep_r = col != (W - 1)
    for j in range(G):
        h1j = h1[:, j * 128:(j + 1) * 128]
        taps = []
        for di in (-W, 0, W):
            for dj in (-1, 0, 1):
                sh = di + dj
                if sh > 0:
                    a = jnp.concatenate(
                        [h1j[sh:], jnp.zeros((sh, 128), jnp.bfloat16)], axis=0)
                elif sh < 0:
                    a = jnp.concatenate(
                        [jnp.zeros((-sh, 128), jnp.bfloat16), h1j[:HW + sh]],
                        axis=0)
                else:
                    a = h1j
                if dj == -1:
                    a = jnp.where(keep_l, a, jnp.bfloat16(0))
                elif dj == 1:
                    a = jnp.where(keep_r, a, jnp.bfloat16(0))
                taps.append(a)
        amat = jnp.concatenate(taps, axis=1)
        acc = jnp.dot(amat, w2_ref[j], preferred_element_type=jnp.float32)
        acc = jnp.maximum(acc + b2_ref[j], 0.0)
        o_ref[0, :, j * 128:(j + 1) * 128] = acc.astype(jnp.bfloat16)


def _avg_kernel(x_ref, o_ref):
    x = x_ref[...].astype(jnp.float32)
    o_ref[...] = jnp.sum(x, axis=1) * (1.0 / x.shape[1])


def _matmul(a, w, bias, residual=None, *, relu):
    M, K = a.shape
    N = w.shape[1]
    tm = _pick_tm(M)
    Mp = _round_up(M, tm)
    if Mp != M:
        a = jnp.pad(a, ((0, Mp - M), (0, 0)))
    tn = 512 if N % 512 == 0 else (256 if N % 256 == 0 else 128)
    bias2 = bias.reshape(1, N)
    grid = (Mp // tm, N // tn)
    in_specs = [
        pl.BlockSpec((tm, K), lambda i, j: (i, 0)),
        pl.BlockSpec((K, tn), lambda i, j: (0, j)),
        pl.BlockSpec((1, tn), lambda i, j: (0, j)),
    ]
    args = [a, w, bias2]
    if residual is not None:
        r = residual
        if Mp != M:
            r = jnp.pad(r, ((0, Mp - M), (0, 0)))
        in_specs.append(pl.BlockSpec((tm, tn), lambda i, j: (i, j)))
        args.append(r)
        body = functools.partial(_mm_res_kernel, relu=relu)
    else:
        body = functools.partial(_mm_kernel, relu=relu)
    out = pl.pallas_call(
        body,
        out_shape=jax.ShapeDtypeStruct((Mp, N), jnp.bfloat16),
        grid=grid,
        in_specs=in_specs,
        out_specs=pl.BlockSpec((tm, tn), lambda i, j: (i, j)),
        compiler_params=_cparams("parallel", "parallel"),
    )(*args)
    return out[:M] if Mp != M else out


def _conv1x1(x, w, b, *, relu, stride=1, residual=None):
    if stride > 1:
        B, H, W, C = x.shape
        z = x.reshape(B, H // 2, 2, W // 2, 2, C)
        x = jnp.transpose(z, (0, 2, 4, 1, 3, 5))[:, 0, 0]
    B, H, W, C = x.shape
    out = _matmul(x.reshape(B * H * W, C), w, b, residual=residual, relu=relu)
    return out.reshape(B, H, W, w.shape[1])


def _fused_conv1_gconv(x, w1, b1, w2, b2):
    B, H, W, C = x.shape
    width = w1.shape[1]
    G = width // 128
    wt = w2.reshape(9, G, 128, G, 128)
    gi = jnp.arange(G)
    wk = wt[:, gi, :, gi, :].reshape(G, 9 * 128, 128)
    out = pl.pallas_call(
        functools.partial(_fused_b_kernel, W=W, G=G),
        out_shape=jax.ShapeDtypeStruct((B, H * W, width), jnp.bfloat16),
        grid=(B,),
        in_specs=[
            pl.BlockSpec((1, H * W, C), lambda b: (b, 0, 0)),
            pl.BlockSpec((C, width), lambda b: (0, 0)),
            pl.BlockSpec((1, width), lambda b: (0, 0)),
            pl.BlockSpec((G, 9 * 128, 128), lambda b: (0, 0, 0)),
            pl.BlockSpec((G, 1, 128), lambda b: (0, 0, 0)),
        ],
        out_specs=pl.BlockSpec((1, H * W, width), lambda b: (b, 0, 0)),
        compiler_params=_cparams("parallel"),
    )(x.reshape(B, H * W, C), w1, b1.reshape(1, width), wk,
      b2.reshape(G, 1, 128))
    return out.reshape(B, H, W, width)


def _tap_views(xp, Ho, Wo, kh, kw, stride):
    views = []
    for i in range(kh):
        for j in range(kw):
            views.append(xp[:, i:i + stride * (Ho - 1) + 1:stride,
                            j:j + stride * (Wo - 1) + 1:stride, :])
    return views


def _parity_views3x3s2(x, pad_val):
    B, H, W, C = x.shape
    Ho, Wo = H // 2, W // 2
    z = x.reshape(B, Ho, 2, Wo, 2, C)
    zz = jnp.transpose(z, (0, 2, 4, 1, 3, 5))
    views = []
    for p, rs in ((1, 1), (0, 0), (1, 0)):
        for q, cs in ((1, 1), (0, 0), (1, 0)):
            v = zz[:, p, q]
            if rs:
                v = jnp.pad(v, ((0, 0), (1, 0), (0, 0), (0, 0)),
                            constant_values=pad_val)[:, :Ho]
            if cs:
                v = jnp.pad(v, ((0, 0), (0, 0), (1, 0), (0, 0)),
                            constant_values=pad_val)[:, :, :Wo]
            views.append(v)
    return views, Ho, Wo


def _gconv3x3(x, w2, b2, *, stride):
    B, H, W, C = x.shape
    G = C // 128
    if stride == 2:
        views, Ho, Wo = _parity_views3x3s2(x, 0.0)
    else:
        xp = jnp.pad(x, ((0, 0), (1, 1), (1, 1), (0, 0)))
        Ho, Wo = H, W
        views = _tap_views(xp, Ho, Wo, 3, 3, 1)
    M = B * Ho * Wo
    a = jnp.stack(views, axis=0).reshape(9, M, C)
    wt = w2.reshape(9, G, 128, G, 128)
    gi = jnp.arange(G)
    wk = wt[:, gi, :, gi, :].reshape(G, 9 * 128, 128)
    bias = b2.reshape(G, 1, 128)
    tm = _pick_tm(M)
    Mp = _round_up(M, tm)
    if Mp != M:
        a = jnp.pad(a, ((0, 0), (0, Mp - M), (0, 0)))
    out = pl.pallas_call(
        _gconv_kernel,
        out_shape=jax.ShapeDtypeStruct((Mp, C), jnp.bfloat16),
        grid=(Mp // tm, G),
        in_specs=[
            pl.BlockSpec((9, tm, 128), lambda i, j: (0, i, j)),
            pl.BlockSpec((1, 9 * 128, 128), lambda i, j: (j, 0, 0)),
            pl.BlockSpec((1, 1, 128), lambda i, j: (j, 0, 0)),
        ],
        out_specs=pl.BlockSpec((tm, 128), lambda i, j: (i, j)),
        compiler_params=_cparams("parallel", "parallel"),
    )(a, wk, bias)
    if Mp != M:
        out = out[:M]
    return out.reshape(B, Ho, Wo, C)


def _stem(x_nchw, w, b):
    x = jnp.transpose(x_nchw, (0, 2, 3, 1)).astype(jnp.bfloat16)
    B, H, W, C = x.shape
    Ho, Wo = H // 2, W // 2
    xs = jnp.pad(x, ((0, 0), (3, 3), (3, 3), (0, 0)))
    Y = xs.reshape(B, Ho + 3, 2, Wo + 3, 2, C)
    Y = jnp.transpose(Y, (0, 1, 3, 2, 4, 5)).reshape(B, Ho + 3, Wo + 3, 4 * C)
    views = []
    for a_ in range(4):
        for b_ in range(4):
            views.append(Y[:, a_:a_ + Ho, b_:b_ + Wo, :])
    M = B * Ho * Wo
    amat = jnp.stack(views, axis=3).reshape(M, 16 * 4 * C)
    Kp = w.shape[0]
    amat = jnp.pad(amat, ((0, 0), (0, Kp - 16 * 4 * C)))
    idx = []
    for a_ in range(4):
        for b_ in range(4):
            for p in range(2):
                for q in range(2):
                    for c in range(C):
                        i, j = 2 * a_ + p, 2 * b_ + q
                        idx.append(((i * 7 + j) * C + c) if i < 7 and j < 7
                                   else Kp - 1)
    idx += [Kp - 1] * (Kp - len(idx))
    w4 = jnp.take(w, jnp.array(idx), axis=0)
    out = _matmul(amat, w4, b, relu=True)
    return out.reshape(B, Ho, Wo, w.shape[1])


def _maxpool3x3s2(x):
    B, H, W, C = x.shape
    views, Ho, Wo = _parity_views3x3s2(x, -jnp.inf)
    M = B * Ho * Wo
    a = jnp.stack(views, axis=0).reshape(9, M, C)
    tm = _pick_tm(M)
    out = pl.pallas_call(
        _max_kernel,
        out_shape=jax.ShapeDtypeStruct((M, C), x.dtype),
        grid=(M // tm,),
        in_specs=[pl.BlockSpec((9, tm, C), lambda i: (0, i, 0))],
        out_specs=pl.BlockSpec((tm, C), lambda i: (i, 0)),
        compiler_params=_cparams("parallel"),
    )(a)
    return out.reshape(B, Ho, Wo, C)


def _avgpool(x):
    B, H, W, C = x.shape
    a = x.reshape(B, H * W, C)
    tc = 256
    out = pl.pallas_call(
        _avg_kernel,
        out_shape=jax.ShapeDtypeStruct((B, C), jnp.float32),
        grid=(C // tc,),
        in_specs=[pl.BlockSpec((B, H * W, tc), lambda j: (0, 0, j))],
        out_specs=pl.BlockSpec((B, tc), lambda j: (0, j)),
        compiler_params=_cparams("parallel"),
    )(a)
    return out


def _bottleneck(x, w1, b1, w2, b2, w3, b3, wd, bd, stride):
    if stride == 1 and x.shape[1] >= 14:
        out = _fused_conv1_gconv(x, w1, b1, w2, b2)
    else:
        out = _conv1x1(x, w1, b1, relu=True)
        out = _gconv3x3(out, w2, b2, stride=stride)
    if wd is not None:
        identity = _conv1x1(x, wd, bd, relu=False, stride=stride)
    else:
        identity = x
    B, H, W, C = out.shape
    res = identity.reshape(B * H * W, identity.shape[-1])
    out = _matmul(out.reshape(B * H * W, C), w3, b3, residual=res, relu=True)
    return out.reshape(B, H, W, w3.shape[1])


def kernel(stem_w, stem_b, s0_b0_w1, s0_b0_b1, s0_b0_w2, s0_b0_b2, s0_b0_w3, s0_b0_b3, s0_b0_wd, s0_b0_bd, s0_b1_w1, s0_b1_b1, s0_b1_w2, s0_b1_b2, s0_b1_w3, s0_b1_b3, s0_b2_w1, s0_b2_b1, s0_b2_w2, s0_b2_b2, s0_b2_w3, s0_b2_b3, s1_b0_w1, s1_b0_b1, s1_b0_w2, s1_b0_b2, s1_b0_w3, s1_b0_b3, s1_b0_wd, s1_b0_bd, s1_b1_w1, s1_b1_b1, s1_b1_w2, s1_b1_b2, s1_b1_w3, s1_b1_b3, s1_b2_w1, s1_b2_b1, s1_b2_w2, s1_b2_b2, s1_b2_w3, s1_b2_b3, s1_b3_w1, s1_b3_b1, s1_b3_w2, s1_b3_b2, s1_b3_w3, s1_b3_b3, s2_b0_w1, s2_b0_b1, s2_b0_w2, s2_b0_b2, s2_b0_w3, s2_b0_b3, s2_b0_wd, s2_b0_bd, s2_b1_w1, s2_b1_b1, s2_b1_w2, s2_b1_b2, s2_b1_w3, s2_b1_b3, s2_b2_w1, s2_b2_b1, s2_b2_w2, s2_b2_b2, s2_b2_w3, s2_b2_b3, s2_b3_w1, s2_b3_b1, s2_b3_w2, s2_b3_b2, s2_b3_w3, s2_b3_b3, s2_b4_w1, s2_b4_b1, s2_b4_w2, s2_b4_b2, s2_b4_w3, s2_b4_b3, s2_b5_w1, s2_b5_b1, s2_b5_w2, s2_b5_b2, s2_b5_w3, s2_b5_b3, s3_b0_w1, s3_b0_b1, s3_b0_w2, s3_b0_b2, s3_b0_w3, s3_b0_b3, s3_b0_wd, s3_b0_bd, s3_b1_w1, s3_b1_b1, s3_b1_w2, s3_b1_b2, s3_b1_w3, s3_b1_b3, s3_b2_w1, s3_b2_b1, s3_b2_w2, s3_b2_b2, s3_b2_w3, s3_b2_b3, x):
    L = dict(locals())
    out = _stem(x, stem_w, stem_b)
    out = _maxpool3x3s2(out)
    for si, cnt in enumerate(_COUNTS):
        for bi in range(cnt):
            stride = 2 if (bi == 0 and si > 0) else 1
            out = _bottleneck(
                out,
                L[f"s{si}_b{bi}_w1"], L[f"s{si}_b{bi}_b1"],
                L[f"s{si}_b{bi}_w2"], L[f"s{si}_b{bi}_b2"],
                L[f"s{si}_b{bi}_w3"], L[f"s{si}_b{bi}_b3"],
                L.get(f"s{si}_b{bi}_wd"), L.get(f"s{si}_b{bi}_bd"),
                stride,
            )
    pool = _avgpool(out)
    return pool.reshape(pool.shape[0], -1, 1, 1)


# speedup vs baseline: 8.1514x; 1.0856x over previous
import functools

import jax
import jax.numpy as jnp
from jax.experimental import pallas as pl
from jax.experimental.pallas import tpu as pltpu

_VMEM = 48 * 1024 * 1024
_COUNTS = (3, 4, 6, 3)


def _round_up(x, m):
    return (x + m - 1) // m * m


def _pick_tm(M):
    for tm in (512, 448, 784, 256, 128, 64, 32, 16, 8):
        if M % tm == 0:
            return tm
    return 256


def _cparams(*sem):
    return pltpu.CompilerParams(dimension_semantics=sem,
                                vmem_limit_bytes=_VMEM)


def _mm_kernel(a_ref, w_ref, b_ref, o_ref, *, relu):
    acc = jnp.dot(a_ref[...], w_ref[...], preferred_element_type=jnp.float32)
    acc = acc + b_ref[...]
    if relu:
        acc = jnp.maximum(acc, 0.0)
    o_ref[...] = acc.astype(o_ref.dtype)


def _mm_res_kernel(a_ref, w_ref, b_ref, r_ref, o_ref, *, relu):
    acc = jnp.dot(a_ref[...], w_ref[...], preferred_element_type=jnp.float32)
    acc = acc + b_ref[...] + r_ref[...].astype(jnp.float32)
    if relu:
        acc = jnp.maximum(acc, 0.0)
    o_ref[...] = acc.astype(o_ref.dtype)


def _gconv_kernel(a_ref, w_ref, b_ref, o_ref):
    a = jnp.concatenate([a_ref[t] for t in range(9)], axis=1)
    acc = jnp.dot(a, w_ref[0], preferred_element_type=jnp.float32)
    acc = jnp.maximum(acc + b_ref[0], 0.0)
    o_ref[...] = acc.astype(o_ref.dtype)


def _max_kernel(x_ref, o_ref):
    o_ref[...] = jnp.max(x_ref[...], axis=0)


def _gconv_bands(h1, W, G, w2_ref, b2_ref):
    HW = h1.shape[0]
    col = jax.lax.broadcasted_iota(jnp.int32, (HW, 128), 0) % W
    keep_l = col != 0
    keep_r = col != (W - 1)
    bands = []
    for j in range(G):
        h1j = h1[:, j * 128:(j + 1) * 128]
        taps = []
        for di in (-W, 0, W):
            for dj in (-1, 0, 1):
                sh = di + dj
                if sh > 0:
                    a = jnp.concatenate(
                        [h1j[sh:], jnp.zeros((sh, 128), jnp.bfloat16)], axis=0)
                elif sh < 0:
                    a = jnp.concatenate(
                        [jnp.zeros((-sh, 128), jnp.bfloat16), h1j[:HW + sh]],
                        axis=0)
                else:
                    a = h1j
                if dj == -1:
                    a = jnp.where(keep_l, a, jnp.bfloat16(0))
                elif dj == 1:
                    a = jnp.where(keep_r, a, jnp.bfloat16(0))
                taps.append(a)
        amat = jnp.concatenate(taps, axis=1)
        acc = jnp.dot(amat, w2_ref[j], preferred_element_type=jnp.float32)
        acc = jnp.maximum(acc + b2_ref[j], 0.0)
        bands.append(acc.astype(jnp.bfloat16))
    return bands


def _fused_b_kernel(x_ref, w1_ref, b1_ref, w2_ref, b2_ref, o_ref, *, W, G):
    xm = x_ref[0]
    h1 = jnp.dot(xm, w1_ref[...], preferred_element_type=jnp.float32)
    h1 = jnp.maximum(h1 + b1_ref[...], 0.0).astype(jnp.bfloat16)
    for j, band in enumerate(_gconv_bands(h1, W, G, w2_ref, b2_ref)):
        o_ref[0, :, j * 128:(j + 1) * 128] = band


def _fused_b3_kernel(x_ref, w1_ref, b1_ref, w2_ref, b2_ref, w3_ref, b3_ref,
                     o_ref, *, W, G):
    xm = x_ref[0]
    h1 = jnp.dot(xm, w1_ref[...], preferred_element_type=jnp.float32)
    h1 = jnp.maximum(h1 + b1_ref[...], 0.0).astype(jnp.bfloat16)
    h2 = jnp.concatenate(_gconv_bands(h1, W, G, w2_ref, b2_ref), axis=1)
    acc = jnp.dot(h2, w3_ref[...], preferred_element_type=jnp.float32)
    acc = acc + b3_ref[...] + xm.astype(jnp.float32)
    o_ref[0] = jnp.maximum(acc, 0.0).astype(jnp.bfloat16)


def _avg_kernel(x_ref, o_ref):
    x = x_ref[...].astype(jnp.float32)
    o_ref[...] = jnp.sum(x, axis=1) * (1.0 / x.shape[1])


def _matmul(a, w, bias, residual=None, *, relu):
    M, K = a.shape
    N = w.shape[1]
    tm = _pick_tm(M)
    Mp = _round_up(M, tm)
    if Mp != M:
        a = jnp.pad(a, ((0, Mp - M), (0, 0)))
    tn = 512 if N % 512 == 0 else (256 if N % 256 == 0 else 128)
    bias2 = bias.reshape(1, N)
    grid = (Mp // tm, N // tn)
    in_specs = [
        pl.BlockSpec((tm, K), lambda i, j: (i, 0)),
        pl.BlockSpec((K, tn), lambda i, j: (0, j)),
        pl.BlockSpec((1, tn), lambda i, j: (0, j)),
    ]
    args = [a, w, bias2]
    if residual is not None:
        r = residual
        if Mp != M:
            r = jnp.pad(r, ((0, Mp - M), (0, 0)))
        in_specs.append(pl.BlockSpec((tm, tn), lambda i, j: (i, j)))
        args.append(r)
        body = functools.partial(_mm_res_kernel, relu=relu)
    else:
        body = functools.partial(_mm_kernel, relu=relu)
    out = pl.pallas_call(
        body,
        out_shape=jax.ShapeDtypeStruct((Mp, N), jnp.bfloat16),
        grid=grid,
        in_specs=in_specs,
        out_specs=pl.BlockSpec((tm, tn), lambda i, j: (i, j)),
        compiler_params=_cparams("parallel", "parallel"),
    )(*args)
    return out[:M] if Mp != M else out


def _conv1x1(x, w, b, *, relu, stride=1, residual=None):
    if stride > 1:
        B, H, W, C = x.shape
        z = x.reshape(B, H // 2, 2, W // 2, 2, C)
        x = jnp.transpose(z, (0, 2, 4, 1, 3, 5))[:, 0, 0]
    B, H, W, C = x.shape
    out = _matmul(x.reshape(B * H * W, C), w, b, residual=residual, relu=relu)
    return out.reshape(B, H, W, w.shape[1])


def _fused_conv1_gconv(x, w1, b1, w2, b2):
    B, H, W, C = x.shape
    width = w1.shape[1]
    G = width // 128
    wt = w2.reshape(9, G, 128, G, 128)
    gi = jnp.arange(G)
    wk = wt[:, gi, :, gi, :].reshape(G, 9 * 128, 128)
    out = pl.pallas_call(
        functools.partial(_fused_b_kernel, W=W, G=G),
        out_shape=jax.ShapeDtypeStruct((B, H * W, width), jnp.bfloat16),
        grid=(B,),
        in_specs=[
            pl.BlockSpec((1, H * W, C), lambda b: (b, 0, 0)),
            pl.BlockSpec((C, width), lambda b: (0, 0)),
            pl.BlockSpec((1, width), lambda b: (0, 0)),
            pl.BlockSpec((G, 9 * 128, 128), lambda b: (0, 0, 0)),
            pl.BlockSpec((G, 1, 128), lambda b: (0, 0, 0)),
        ],
        out_specs=pl.BlockSpec((1, H * W, width), lambda b: (b, 0, 0)),
        compiler_params=_cparams("parallel"),
    )(x.reshape(B, H * W, C), w1, b1.reshape(1, width), wk,
      b2.reshape(G, 1, 128))
    return out.reshape(B, H, W, width)


def _fused_bottleneck(x, w1, b1, w2, b2, w3, b3):
    B, H, W, C = x.shape
    width = w1.shape[1]
    N3 = w3.shape[1]
    G = width // 128
    wt = w2.reshape(9, G, 128, G, 128)
    gi = jnp.arange(G)
    wk = wt[:, gi, :, gi, :].reshape(G, 9 * 128, 128)
    out = pl.pallas_call(
        functools.partial(_fused_b3_kernel, W=W, G=G),
        out_shape=jax.ShapeDtypeStruct((B, H * W, N3), jnp.bfloat16),
        grid=(B,),
        in_specs=[
            pl.BlockSpec((1, H * W, C), lambda b: (b, 0, 0)),
            pl.BlockSpec((C, width), lambda b: (0, 0)),
            pl.BlockSpec((1, width), lambda b: (0, 0)),
            pl.BlockSpec((G, 9 * 128, 128), lambda b: (0, 0, 0)),
            pl.BlockSpec((G, 1, 128), lambda b: (0, 0, 0)),
            pl.BlockSpec((width, N3), lambda b: (0, 0)),
            pl.BlockSpec((1, N3), lambda b: (0, 0)),
        ],
        out_specs=pl.BlockSpec((1, H * W, N3), lambda b: (b, 0, 0)),
        compiler_params=_cparams("parallel"),
    )(x.reshape(B, H * W, C), w1, b1.reshape(1, width), wk,
      b2.reshape(G, 1, 128), w3, b3.reshape(1, N3))
    return out.reshape(B, H, W, N3)


def _tap_views(xp, Ho, Wo, kh, kw, stride):
    views = []
    for i in range(kh):
        for j in range(kw):
            views.append(xp[:, i:i + stride * (Ho - 1) + 1:stride,
                            j:j + stride * (Wo - 1) + 1:stride, :])
    return views


def _parity_views3x3s2(x, pad_val):
    B, H, W, C = x.shape
    Ho, Wo = H // 2, W // 2
    z = x.reshape(B, Ho, 2, Wo, 2, C)
    zz = jnp.transpose(z, (0, 2, 4, 1, 3, 5))
    views = []
    for p, rs in ((1, 1), (0, 0), (1, 0)):
        for q, cs in ((1, 1), (0, 0), (1, 0)):
            v = zz[:, p, q]
            if rs:
                v = jnp.pad(v, ((0, 0), (1, 0), (0, 0), (0, 0)),
                            constant_values=pad_val)[:, :Ho]
            if cs:
                v = jnp.pad(v, ((0, 0), (0, 0), (1, 0), (0, 0)),
                            constant_values=pad_val)[:, :, :Wo]
            views.append(v)
    return views, Ho, Wo


def _gconv3x3(x, w2, b2, *, stride):
    B, H, W, C = x.shape
    G = C // 128
    if stride == 2:
        views, Ho, Wo = _parity_views3x3s2(x, 0.0)
    else:
        xp = jnp.pad(x, ((0, 0), (1, 1), (1, 1), (0, 0)))
        Ho, Wo = H, W
        views = _tap_views(xp, Ho, Wo, 3, 3, 1)
    M = B * Ho * Wo
    a = jnp.stack(views, axis=0).reshape(9, M, C)
    wt = w2.reshape(9, G, 128, G, 128)
    gi = jnp.arange(G)
    wk = wt[:, gi, :, gi, :].reshape(G, 9 * 128, 128)
    bias = b2.reshape(G, 1, 128)
    tm = _pick_tm(M)
    Mp = _round_up(M, tm)
    if Mp != M:
        a = jnp.pad(a, ((0, 0), (0, Mp - M), (0, 0)))
    out = pl.pallas_call(
        _gconv_kernel,
        out_shape=jax.ShapeDtypeStruct((Mp, C), jnp.bfloat16),
        grid=(Mp // tm, G),
        in_specs=[
            pl.BlockSpec((9, tm, 128), lambda i, j: (0, i, j)),
            pl.BlockSpec((1, 9 * 128, 128), lambda i, j: (j, 0, 0)),
            pl.BlockSpec((1, 1, 128), lambda i, j: (j, 0, 0)),
        ],
        out_specs=pl.BlockSpec((tm, 128), lambda i, j: (i, j)),
        compiler_params=_cparams("parallel", "parallel"),
    )(a, wk, bias)
    if Mp != M:
        out = out[:M]
    return out.reshape(B, Ho, Wo, C)


def _stem(x_nchw, w, b):
    x = jnp.transpose(x_nchw, (0, 2, 3, 1)).astype(jnp.bfloat16)
    B, H, W, C = x.shape
    Ho, Wo = H // 2, W // 2
    xs = jnp.pad(x, ((0, 0), (3, 3), (3, 3), (0, 0)))
    Y = xs.reshape(B, Ho + 3, 2, Wo + 3, 2, C)
    Y = jnp.transpose(Y, (0, 1, 3, 2, 4, 5)).reshape(B, Ho + 3, Wo + 3, 4 * C)
    views = []
    for a_ in range(4):
        for b_ in range(4):
            views.append(Y[:, a_:a_ + Ho, b_:b_ + Wo, :])
    M = B * Ho * Wo
    amat = jnp.stack(views, axis=3).reshape(M, 16 * 4 * C)
    Kp = w.shape[0]
    amat = jnp.pad(amat, ((0, 0), (0, Kp - 16 * 4 * C)))
    idx = []
    for a_ in range(4):
        for b_ in range(4):
            for p in range(2):
                for q in range(2):
                    for c in range(C):
                        i, j = 2 * a_ + p, 2 * b_ + q
                        idx.append(((i * 7 + j) * C + c) if i < 7 and j < 7
                                   else Kp - 1)
    idx += [Kp - 1] * (Kp - len(idx))
    w4 = jnp.take(w, jnp.array(idx), axis=0)
    out = _matmul(amat, w4, b, relu=True)
    return out.reshape(B, Ho, Wo, w.shape[1])


def _maxpool3x3s2(x):
    B, H, W, C = x.shape
    views, Ho, Wo = _parity_views3x3s2(x, -jnp.inf)
    M = B * Ho * Wo
    a = jnp.stack(views, axis=0).reshape(9, M, C)
    tm = _pick_tm(M)
    out = pl.pallas_call(
        _max_kernel,
        out_shape=jax.ShapeDtypeStruct((M, C), x.dtype),
        grid=(M // tm,),
        in_specs=[pl.BlockSpec((9, tm, C), lambda i: (0, i, 0))],
        out_specs=pl.BlockSpec((tm, C), lambda i: (i, 0)),
        compiler_params=_cparams("parallel"),
    )(a)
    return out.reshape(B, Ho, Wo, C)


def _avgpool(x):
    B, H, W, C = x.shape
    a = x.reshape(B, H * W, C)
    tc = 256
    out = pl.pallas_call(
        _avg_kernel,
        out_shape=jax.ShapeDtypeStruct((B, C), jnp.float32),
        grid=(C // tc,),
        in_specs=[pl.BlockSpec((B, H * W, tc), lambda j: (0, 0, j))],
        out_specs=pl.BlockSpec((B, tc), lambda j: (0, j)),
        compiler_params=_cparams("parallel"),
    )(a)
    return out


def _bottleneck(x, w1, b1, w2, b2, w3, b3, wd, bd, stride):
    if stride == 1 and x.shape[1] >= 14:
        if wd is None:
            return _fused_bottleneck(x, w1, b1, w2, b2, w3, b3)
        out = _fused_conv1_gconv(x, w1, b1, w2, b2)
    else:
        out = _conv1x1(x, w1, b1, relu=True)
        out = _gconv3x3(out, w2, b2, stride=stride)
    if wd is not None:
        identity = _conv1x1(x, wd, bd, relu=False, stride=stride)
    else:
        identity = x
    B, H, W, C = out.shape
    res = identity.reshape(B * H * W, identity.shape[-1])
    out = _matmul(out.reshape(B * H * W, C), w3, b3, residual=res, relu=True)
    return out.reshape(B, H, W, w3.shape[1])


def kernel(stem_w, stem_b, s0_b0_w1, s0_b0_b1, s0_b0_w2, s0_b0_b2, s0_b0_w3, s0_b0_b3, s0_b0_wd, s0_b0_bd, s0_b1_w1, s0_b1_b1, s0_b1_w2, s0_b1_b2, s0_b1_w3, s0_b1_b3, s0_b2_w1, s0_b2_b1, s0_b2_w2, s0_b2_b2, s0_b2_w3, s0_b2_b3, s1_b0_w1, s1_b0_b1, s1_b0_w2, s1_b0_b2, s1_b0_w3, s1_b0_b3, s1_b0_wd, s1_b0_bd, s1_b1_w1, s1_b1_b1, s1_b1_w2, s1_b1_b2, s1_b1_w3, s1_b1_b3, s1_b2_w1, s1_b2_b1, s1_b2_w2, s1_b2_b2, s1_b2_w3, s1_b2_b3, s1_b3_w1, s1_b3_b1, s1_b3_w2, s1_b3_b2, s1_b3_w3, s1_b3_b3, s2_b0_w1, s2_b0_b1, s2_b0_w2, s2_b0_b2, s2_b0_w3, s2_b0_b3, s2_b0_wd, s2_b0_bd, s2_b1_w1, s2_b1_b1, s2_b1_w2, s2_b1_b2, s2_b1_w3, s2_b1_b3, s2_b2_w1, s2_b2_b1, s2_b2_w2, s2_b2_b2, s2_b2_w3, s2_b2_b3, s2_b3_w1, s2_b3_b1, s2_b3_w2, s2_b3_b2, s2_b3_w3, s2_b3_b3, s2_b4_w1, s2_b4_b1, s2_b4_w2, s2_b4_b2, s2_b4_w3, s2_b4_b3, s2_b5_w1, s2_b5_b1, s2_b5_w2, s2_b5_b2, s2_b5_w3, s2_b5_b3, s3_b0_w1, s3_b0_b1, s3_b0_w2, s3_b0_b2, s3_b0_w3, s3_b0_b3, s3_b0_wd, s3_b0_bd, s3_b1_w1, s3_b1_b1, s3_b1_w2, s3_b1_b2, s3_b1_w3, s3_b1_b3, s3_b2_w1, s3_b2_b1, s3_b2_w2, s3_b2_b2, s3_b2_w3, s3_b2_b3, x):
    L = dict(locals())
    out = _stem(x, stem_w, stem_b)
    out = _maxpool3x3s2(out)
    for si, cnt in enumerate(_COUNTS):
        for bi in range(cnt):
            stride = 2 if (bi == 0 and si > 0) else 1
            out = _bottleneck(
                out,
                L[f"s{si}_b{bi}_w1"], L[f"s{si}_b{bi}_b1"],
                L[f"s{si}_b{bi}_w2"], L[f"s{si}_b{bi}_b2"],
                L[f"s{si}_b{bi}_w3"], L[f"s{si}_b{bi}_b3"],
                L.get(f"s{si}_b{bi}_wd"), L.get(f"s{si}_b{bi}_bd"),
                stride,
            )
    pool = _avgpool(out)
    return pool.reshape(pool.shape[0], -1, 1, 1)


# speedup vs baseline: 8.7302x; 1.0710x over previous
import functools

import jax
import jax.numpy as jnp
from jax.experimental import pallas as pl
from jax.experimental.pallas import tpu as pltpu

_VMEM = 48 * 1024 * 1024
_COUNTS = (3, 4, 6, 3)


def _round_up(x, m):
    return (x + m - 1) // m * m


def _pick_tm(M):
    for tm in (512, 448, 784, 256, 128, 64, 32, 16, 8):
        if M % tm == 0:
            return tm
    return 256


def _cparams(*sem):
    return pltpu.CompilerParams(dimension_semantics=sem,
                                vmem_limit_bytes=_VMEM)


def _mm_kernel(a_ref, w_ref, b_ref, o_ref, *, relu):
    acc = jnp.dot(a_ref[...], w_ref[...], preferred_element_type=jnp.float32)
    acc = acc + b_ref[...]
    if relu:
        acc = jnp.maximum(acc, 0.0)
    o_ref[...] = acc.astype(o_ref.dtype)


def _mm_res_kernel(a_ref, w_ref, b_ref, r_ref, o_ref, *, relu):
    acc = jnp.dot(a_ref[...], w_ref[...], preferred_element_type=jnp.float32)
    acc = acc + b_ref[...] + r_ref[...].astype(jnp.float32)
    if relu:
        acc = jnp.maximum(acc, 0.0)
    o_ref[...] = acc.astype(o_ref.dtype)


def _gconv_kernel(a_ref, w_ref, b_ref, o_ref):
    a = jnp.concatenate([a_ref[t] for t in range(9)], axis=1)
    acc = jnp.dot(a, w_ref[0], preferred_element_type=jnp.float32)
    acc = jnp.maximum(acc + b_ref[0], 0.0)
    o_ref[...] = acc.astype(o_ref.dtype)


def _max_kernel(x_ref, o_ref):
    o_ref[...] = jnp.max(x_ref[...], axis=0)


def _maxpool_kernel(z_ref, o_ref, *, HW, W):
    ninf = jnp.bfloat16(-jnp.inf)
    col = jax.lax.broadcasted_iota(jnp.int32, (HW, z_ref.shape[-1]), 0) % W
    keep_l = col != 0
    planes = {(p, q): z_ref[0, p, q].reshape(HW, z_ref.shape[-1])
              for p in range(2) for q in range(2)}
    acc = None
    for p, rs in ((1, 1), (0, 0), (1, 0)):
        for q, cs in ((1, 1), (0, 0), (1, 0)):
            v = planes[(p, q)]
            sh = rs * W + cs
            if sh:
                v = jnp.concatenate(
                    [jnp.full((sh, v.shape[-1]), ninf), v[:HW - sh]], axis=0)
            if cs:
                v = jnp.where(keep_l, v, ninf)
            acc = v if acc is None else jnp.maximum(acc, v)
    o_ref[0] = acc


def _gconv_bands(h1, W, G, w2_ref, b2_ref):
    HW = h1.shape[0]
    col = jax.lax.broadcasted_iota(jnp.int32, (HW, 128), 0) % W
    keep_l = col != 0
    keep_r = col != (W - 1)
    bands = []
    for j in range(G):
        h1j = h1[:, j * 128:(j + 1) * 128]
        taps = []
        for di in (-W, 0, W):
            for dj in (-1, 0, 1):
                sh = di + dj
                if sh > 0:
                    a = jnp.concatenate(
                        [h1j[sh:], jnp.zeros((sh, 128), jnp.bfloat16)], axis=0)
                elif sh < 0:
                    a = jnp.concatenate(
                        [jnp.zeros((-sh, 128), jnp.bfloat16), h1j[:HW + sh]],
                        axis=0)
                else:
                    a = h1j
                if dj == -1:
                    a = jnp.where(keep_l, a, jnp.bfloat16(0))
                elif dj == 1:
                    a = jnp.where(keep_r, a, jnp.bfloat16(0))
                taps.append(a)
        amat = jnp.concatenate(taps, axis=1)
        acc = jnp.dot(amat, w2_ref[j], preferred_element_type=jnp.float32)
        acc = jnp.maximum(acc + b2_ref[j], 0.0)
        bands.append(acc.astype(jnp.bfloat16))
    return bands


def _fused_b_kernel(x_ref, w1_ref, b1_ref, w2_ref, b2_ref, o_ref, *, W, G):
    xm = x_ref[0]
    h1 = jnp.dot(xm, w1_ref[...], preferred_element_type=jnp.float32)
    h1 = jnp.maximum(h1 + b1_ref[...], 0.0).astype(jnp.bfloat16)
    for j, band in enumerate(_gconv_bands(h1, W, G, w2_ref, b2_ref)):
        o_ref[0, :, j * 128:(j + 1) * 128] = band


def _fused_b3_kernel(x_ref, w1_ref, b1_ref, w2_ref, b2_ref, w3_ref, b3_ref,
                     o_ref, *, W, G):
    xm = x_ref[0]
    h1 = jnp.dot(xm, w1_ref[...], preferred_element_type=jnp.float32)
    h1 = jnp.maximum(h1 + b1_ref[...], 0.0).astype(jnp.bfloat16)
    h2 = jnp.concatenate(_gconv_bands(h1, W, G, w2_ref, b2_ref), axis=1)
    acc = jnp.dot(h2, w3_ref[...], preferred_element_type=jnp.float32)
    acc = acc + b3_ref[...] + xm.astype(jnp.float32)
    o_ref[0] = jnp.maximum(acc, 0.0).astype(jnp.bfloat16)


def _avg_kernel(x_ref, o_ref):
    x = x_ref[...].astype(jnp.float32)
    o_ref[...] = jnp.sum(x, axis=1) * (1.0 / x.shape[1])


def _matmul(a, w, bias, residual=None, *, relu):
    M, K = a.shape
    N = w.shape[1]
    tm = _pick_tm(M)
    Mp = _round_up(M, tm)
    if Mp != M:
        a = jnp.pad(a, ((0, Mp - M), (0, 0)))
    tn = 512 if N % 512 == 0 else (256 if N % 256 == 0 else 128)
    bias2 = bias.reshape(1, N)
    grid = (Mp // tm, N // tn)
    in_specs = [
        pl.BlockSpec((tm, K), lambda i, j: (i, 0)),
        pl.BlockSpec((K, tn), lambda i, j: (0, j)),
        pl.BlockSpec((1, tn), lambda i, j: (0, j)),
    ]
    args = [a, w, bias2]
    if residual is not None:
        r = residual
        if Mp != M:
            r = jnp.pad(r, ((0, Mp - M), (0, 0)))
        in_specs.append(pl.BlockSpec((tm, tn), lambda i, j: (i, j)))
        args.append(r)
        body = functools.partial(_mm_res_kernel, relu=relu)
    else:
        body = functools.partial(_mm_kernel, relu=relu)
    out = pl.pallas_call(
        body,
        out_shape=jax.ShapeDtypeStruct((Mp, N), jnp.bfloat16),
        grid=grid,
        in_specs=in_specs,
        out_specs=pl.BlockSpec((tm, tn), lambda i, j: (i, j)),
        compiler_params=_cparams("parallel", "parallel"),
    )(*args)
    return out[:M] if Mp != M else out


def _conv1x1(x, w, b, *, relu, stride=1, residual=None):
    if stride > 1:
        B, H, W, C = x.shape
        z = x.reshape(B, H // 2, 2, W // 2, 2, C)
        x = jnp.transpose(z, (0, 2, 4, 1, 3, 5))[:, 0, 0]
    B, H, W, C = x.shape
    out = _matmul(x.reshape(B * H * W, C), w, b, residual=residual, relu=relu)
    return out.reshape(B, H, W, w.shape[1])


def _fused_conv1_gconv(x, w1, b1, w2, b2):
    B, H, W, C = x.shape
    width = w1.shape[1]
    G = width // 128
    wt = w2.reshape(9, G, 128, G, 128)
    gi = jnp.arange(G)
    wk = wt[:, gi, :, gi, :].reshape(G, 9 * 128, 128)
    out = pl.pallas_call(
        functools.partial(_fused_b_kernel, W=W, G=G),
        out_shape=jax.ShapeDtypeStruct((B, H * W, width), jnp.bfloat16),
        grid=(B,),
        in_specs=[
            pl.BlockSpec((1, H * W, C), lambda b: (b, 0, 0)),
            pl.BlockSpec((C, width), lambda b: (0, 0)),
            pl.BlockSpec((1, width), lambda b: (0, 0)),
            pl.BlockSpec((G, 9 * 128, 128), lambda b: (0, 0, 0)),
            pl.BlockSpec((G, 1, 128), lambda b: (0, 0, 0)),
        ],
        out_specs=pl.BlockSpec((1, H * W, width), lambda b: (b, 0, 0)),
        compiler_params=_cparams("parallel"),
    )(x.reshape(B, H * W, C), w1, b1.reshape(1, width), wk,
      b2.reshape(G, 1, 128))
    return out.reshape(B, H, W, width)


def _fused_bottleneck(x, w1, b1, w2, b2, w3, b3):
    B, H, W, C = x.shape
    width = w1.shape[1]
    N3 = w3.shape[1]
    G = width // 128
    wt = w2.reshape(9, G, 128, G, 128)
    gi = jnp.arange(G)
    wk = wt[:, gi, :, gi, :].reshape(G, 9 * 128, 128)
    out = pl.pallas_call(
        functools.partial(_fused_b3_kernel, W=W, G=G),
        out_shape=jax.ShapeDtypeStruct((B, H * W, N3), jnp.bfloat16),
        grid=(B,),
        in_specs=[
            pl.BlockSpec((1, H * W, C), lambda b: (b, 0, 0)),
            pl.BlockSpec((C, width), lambda b: (0, 0)),
            pl.BlockSpec((1, width), lambda b: (0, 0)),
            pl.BlockSpec((G, 9 * 128, 128), lambda b: (0, 0, 0)),
            pl.BlockSpec((G, 1, 128), lambda b: (0, 0, 0)),
            pl.BlockSpec((width, N3), lambda b: (0, 0)),
            pl.BlockSpec((1, N3), lambda b: (0, 0)),
        ],
        out_specs=pl.BlockSpec((1, H * W, N3), lambda b: (b, 0, 0)),
        compiler_params=_cparams("parallel"),
    )(x.reshape(B, H * W, C), w1, b1.reshape(1, width), wk,
      b2.reshape(G, 1, 128), w3, b3.reshape(1, N3))
    return out.reshape(B, H, W, N3)


def _tap_views(xp, Ho, Wo, kh, kw, stride):
    views = []
    for i in range(kh):
        for j in range(kw):
            views.append(xp[:, i:i + stride * (Ho - 1) + 1:stride,
                            j:j + stride * (Wo - 1) + 1:stride, :])
    return views


def _parity_views3x3s2(x, pad_val):
    B, H, W, C = x.shape
    Ho, Wo = H // 2, W // 2
    z = x.reshape(B, Ho, 2, Wo, 2, C)
    zz = jnp.transpose(z, (0, 2, 4, 1, 3, 5))
    views = []
    for p, rs in ((1, 1), (0, 0), (1, 0)):
        for q, cs in ((1, 1), (0, 0), (1, 0)):
            v = zz[:, p, q]
            if rs:
                v = jnp.pad(v, ((0, 0), (1, 0), (0, 0), (0, 0)),
                            constant_values=pad_val)[:, :Ho]
            if cs:
                v = jnp.pad(v, ((0, 0), (0, 0), (1, 0), (0, 0)),
                            constant_values=pad_val)[:, :, :Wo]
            views.append(v)
    return views, Ho, Wo


def _gconv3x3(x, w2, b2, *, stride):
    B, H, W, C = x.shape
    G = C // 128
    if stride == 2:
        views, Ho, Wo = _parity_views3x3s2(x, 0.0)
    else:
        xp = jnp.pad(x, ((0, 0), (1, 1), (1, 1), (0, 0)))
        Ho, Wo = H, W
        views = _tap_views(xp, Ho, Wo, 3, 3, 1)
    M = B * Ho * Wo
    a = jnp.stack(views, axis=0).reshape(9, M, C)
    wt = w2.reshape(9, G, 128, G, 128)
    gi = jnp.arange(G)
    wk = wt[:, gi, :, gi, :].reshape(G, 9 * 128, 128)
    bias = b2.reshape(G, 1, 128)
    tm = _pick_tm(M)
    Mp = _round_up(M, tm)
    if Mp != M:
        a = jnp.pad(a, ((0, 0), (0, Mp - M), (0, 0)))
    out = pl.pallas_call(
        _gconv_kernel,
        out_shape=jax.ShapeDtypeStruct((Mp, C), jnp.bfloat16),
        grid=(Mp // tm, G),
        in_specs=[
            pl.BlockSpec((9, tm, 128), lambda i, j: (0, i, j)),
            pl.BlockSpec((1, 9 * 128, 128), lambda i, j: (j, 0, 0)),
            pl.BlockSpec((1, 1, 128), lambda i, j: (j, 0, 0)),
        ],
        out_specs=pl.BlockSpec((tm, 128), lambda i, j: (i, j)),
        compiler_params=_cparams("parallel", "parallel"),
    )(a, wk, bias)
    if Mp != M:
        out = out[:M]
    return out.reshape(B, Ho, Wo, C)


def _stem(x_nchw, w, b):
    x = jnp.transpose(x_nchw, (0, 2, 3, 1)).astype(jnp.bfloat16)
    B, H, W, C = x.shape
    Ho, Wo = H // 2, W // 2
    xs = jnp.pad(x, ((0, 0), (3, 3), (3, 3), (0, 0)))
    Y = xs.reshape(B, Ho + 3, 2, Wo + 3, 2, C)
    Y = jnp.transpose(Y, (0, 1, 3, 2, 4, 5)).reshape(B, Ho + 3, Wo + 3, 4 * C)
    views = []
    for a_ in range(4):
        for b_ in range(4):
            views.append(Y[:, a_:a_ + Ho, b_:b_ + Wo, :])
    M = B * Ho * Wo
    amat = jnp.stack(views, axis=3).reshape(M, 16 * 4 * C)
    Kp = w.shape[0]
    amat = jnp.pad(amat, ((0, 0), (0, Kp - 16 * 4 * C)))
    idx = []
    for a_ in range(4):
        for b_ in range(4):
            for p in range(2):
                for q in range(2):
                    for c in range(C):
                        i, j = 2 * a_ + p, 2 * b_ + q
                        idx.append(((i * 7 + j) * C + c) if i < 7 and j < 7
                                   else Kp - 1)
    idx += [Kp - 1] * (Kp - len(idx))
    w4 = jnp.take(w, jnp.array(idx), axis=0)
    out = _matmul(amat, w4, b, relu=True)
    return out.reshape(B, Ho, Wo, w.shape[1])


def _maxpool3x3s2(x):
    B, H, W, C = x.shape
    Ho, Wo = H // 2, W // 2
    z = x.reshape(B, Ho, 2, Wo, 2, C)
    zz = jnp.transpose(z, (0, 2, 4, 1, 3, 5))
    out = pl.pallas_call(
        functools.partial(_maxpool_kernel, HW=Ho * Wo, W=Wo),
        out_shape=jax.ShapeDtypeStruct((B, Ho * Wo, C), x.dtype),
        grid=(B,),
        in_specs=[pl.BlockSpec((1, 2, 2, Ho, Wo, C),
                               lambda b: (b, 0, 0, 0, 0, 0))],
        out_specs=pl.BlockSpec((1, Ho * Wo, C), lambda b: (b, 0, 0)),
        compiler_params=_cparams("parallel"),
    )(zz)
    return out.reshape(B, Ho, Wo, C)


def _avgpool(x):
    B, H, W, C = x.shape
    a = x.reshape(B, H * W, C)
    tc = 256
    out = pl.pallas_call(
        _avg_kernel,
        out_shape=jax.ShapeDtypeStruct((B, C), jnp.float32),
        grid=(C // tc,),
        in_specs=[pl.BlockSpec((B, H * W, tc), lambda j: (0, 0, j))],
        out_specs=pl.BlockSpec((B, tc), lambda j: (0, j)),
        compiler_params=_cparams("parallel"),
    )(a)
    return out


def _bottleneck(x, w1, b1, w2, b2, w3, b3, wd, bd, stride):
    if stride == 1 and x.shape[1] >= 14:
        if wd is None:
            return _fused_bottleneck(x, w1, b1, w2, b2, w3, b3)
        out = _fused_conv1_gconv(x, w1, b1, w2, b2)
    else:
        out = _conv1x1(x, w1, b1, relu=True)
        out = _gconv3x3(out, w2, b2, stride=stride)
    if wd is not None:
        identity = _conv1x1(x, wd, bd, relu=False, stride=stride)
    else:
        identity = x
    B, H, W, C = out.shape
    res = identity.reshape(B * H * W, identity.shape[-1])
    out = _matmul(out.reshape(B * H * W, C), w3, b3, residual=res, relu=True)
    return out.reshape(B, H, W, w3.shape[1])


def kernel(stem_w, stem_b, s0_b0_w1, s0_b0_b1, s0_b0_w2, s0_b0_b2, s0_b0_w3, s0_b0_b3, s0_b0_wd, s0_b0_bd, s0_b1_w1, s0_b1_b1, s0_b1_w2, s0_b1_b2, s0_b1_w3, s0_b1_b3, s0_b2_w1, s0_b2_b1, s0_b2_w2, s0_b2_b2, s0_b2_w3, s0_b2_b3, s1_b0_w1, s1_b0_b1, s1_b0_w2, s1_b0_b2, s1_b0_w3, s1_b0_b3, s1_b0_wd, s1_b0_bd, s1_b1_w1, s1_b1_b1, s1_b1_w2, s1_b1_b2, s1_b1_w3, s1_b1_b3, s1_b2_w1, s1_b2_b1, s1_b2_w2, s1_b2_b2, s1_b2_w3, s1_b2_b3, s1_b3_w1, s1_b3_b1, s1_b3_w2, s1_b3_b2, s1_b3_w3, s1_b3_b3, s2_b0_w1, s2_b0_b1, s2_b0_w2, s2_b0_b2, s2_b0_w3, s2_b0_b3, s2_b0_wd, s2_b0_bd, s2_b1_w1, s2_b1_b1, s2_b1_w2, s2_b1_b2, s2_b1_w3, s2_b1_b3, s2_b2_w1, s2_b2_b1, s2_b2_w2, s2_b2_b2, s2_b2_w3, s2_b2_b3, s2_b3_w1, s2_b3_b1, s2_b3_w2, s2_b3_b2, s2_b3_w3, s2_b3_b3, s2_b4_w1, s2_b4_b1, s2_b4_w2, s2_b4_b2, s2_b4_w3, s2_b4_b3, s2_b5_w1, s2_b5_b1, s2_b5_w2, s2_b5_b2, s2_b5_w3, s2_b5_b3, s3_b0_w1, s3_b0_b1, s3_b0_w2, s3_b0_b2, s3_b0_w3, s3_b0_b3, s3_b0_wd, s3_b0_bd, s3_b1_w1, s3_b1_b1, s3_b1_w2, s3_b1_b2, s3_b1_w3, s3_b1_b3, s3_b2_w1, s3_b2_b1, s3_b2_w2, s3_b2_b2, s3_b2_w3, s3_b2_b3, x):
    L = dict(locals())
    out = _stem(x, stem_w, stem_b)
    out = _maxpool3x3s2(out)
    for si, cnt in enumerate(_COUNTS):
        for bi in range(cnt):
            stride = 2 if (bi == 0 and si > 0) else 1
            out = _bottleneck(
                out,
                L[f"s{si}_b{bi}_w1"], L[f"s{si}_b{bi}_b1"],
                L[f"s{si}_b{bi}_w2"], L[f"s{si}_b{bi}_b2"],
                L[f"s{si}_b{bi}_w3"], L[f"s{si}_b{bi}_b3"],
                L.get(f"s{si}_b{bi}_wd"), L.get(f"s{si}_b{bi}_bd"),
                stride,
            )
    pool = _avgpool(out)
    return pool.reshape(pool.shape[0], -1, 1, 1)


# speedup vs baseline: 8.8900x; 1.0183x over previous
import functools

import jax
import jax.numpy as jnp
from jax.experimental import pallas as pl
from jax.experimental.pallas import tpu as pltpu

_VMEM = 48 * 1024 * 1024
_COUNTS = (3, 4, 6, 3)


def _round_up(x, m):
    return (x + m - 1) // m * m


def _pick_tm(M):
    for tm in (512, 448, 784, 256, 128, 64, 32, 16, 8):
        if M % tm == 0:
            return tm
    return 256


def _cparams(*sem):
    return pltpu.CompilerParams(dimension_semantics=sem,
                                vmem_limit_bytes=_VMEM)


def _mm_kernel(a_ref, w_ref, b_ref, o_ref, *, relu):
    acc = jnp.dot(a_ref[...], w_ref[...], preferred_element_type=jnp.float32)
    acc = acc + b_ref[...]
    if relu:
        acc = jnp.maximum(acc, 0.0)
    o_ref[...] = acc.astype(o_ref.dtype)


def _mm_res_kernel(a_ref, w_ref, b_ref, r_ref, o_ref, *, relu):
    acc = jnp.dot(a_ref[...], w_ref[...], preferred_element_type=jnp.float32)
    acc = acc + b_ref[...] + r_ref[...].astype(jnp.float32)
    if relu:
        acc = jnp.maximum(acc, 0.0)
    o_ref[...] = acc.astype(o_ref.dtype)


def _gconv_kernel(a_ref, w_ref, b_ref, o_ref):
    a = jnp.concatenate([a_ref[t] for t in range(9)], axis=1)
    acc = jnp.dot(a, w_ref[0], preferred_element_type=jnp.float32)
    acc = jnp.maximum(acc + b_ref[0], 0.0)
    o_ref[...] = acc.astype(o_ref.dtype)


def _max_kernel(x_ref, o_ref):
    o_ref[...] = jnp.max(x_ref[...], axis=0)


def _maxpool_kernel(z_ref, o_ref, *, HW, W):
    ninf = jnp.bfloat16(-jnp.inf)
    col = jax.lax.broadcasted_iota(jnp.int32, (HW, z_ref.shape[-1]), 0) % W
    keep_l = col != 0
    planes = {(p, q): z_ref[0, p, q].reshape(HW, z_ref.shape[-1])
              for p in range(2) for q in range(2)}
    acc = None
    for p, rs in ((1, 1), (0, 0), (1, 0)):
        for q, cs in ((1, 1), (0, 0), (1, 0)):
            v = planes[(p, q)]
            sh = rs * W + cs
            if sh:
                v = jnp.concatenate(
                    [jnp.full((sh, v.shape[-1]), ninf), v[:HW - sh]], axis=0)
            if cs:
                v = jnp.where(keep_l, v, ninf)
            acc = v if acc is None else jnp.maximum(acc, v)
    o_ref[0] = acc


def _gconv_bands(h1, W, G, w2_ref, b2_ref):
    HW = h1.shape[0]
    col = jax.lax.broadcasted_iota(jnp.int32, (HW, 128), 0) % W
    keep_l = col != 0
    keep_r = col != (W - 1)
    bands = []
    for j in range(G):
        h1j = h1[:, j * 128:(j + 1) * 128]
        taps = []
        for di in (-W, 0, W):
            for dj in (-1, 0, 1):
                sh = di + dj
                if sh > 0:
                    a = jnp.concatenate(
                        [h1j[sh:], jnp.zeros((sh, 128), jnp.bfloat16)], axis=0)
                elif sh < 0:
                    a = jnp.concatenate(
                        [jnp.zeros((-sh, 128), jnp.bfloat16), h1j[:HW + sh]],
                        axis=0)
                else:
                    a = h1j
                if dj == -1:
                    a = jnp.where(keep_l, a, jnp.bfloat16(0))
                elif dj == 1:
                    a = jnp.where(keep_r, a, jnp.bfloat16(0))
                taps.append(a)
        amat = jnp.concatenate(taps, axis=1)
        acc = jnp.dot(amat, w2_ref[j], preferred_element_type=jnp.float32)
        acc = jnp.maximum(acc + b2_ref[j], 0.0)
        bands.append(acc.astype(jnp.bfloat16))
    return bands


def _fused_b_kernel(x_ref, w1_ref, b1_ref, w2_ref, b2_ref, o_ref, *, W, G):
    xm = x_ref[0]
    h1 = jnp.dot(xm, w1_ref[...], preferred_element_type=jnp.float32)
    h1 = jnp.maximum(h1 + b1_ref[...], 0.0).astype(jnp.bfloat16)
    for j, band in enumerate(_gconv_bands(h1, W, G, w2_ref, b2_ref)):
        o_ref[0, :, j * 128:(j + 1) * 128] = band


def _fused_b3_kernel(x_ref, w1_ref, b1_ref, w2_ref, b2_ref, w3_ref, b3_ref,
                     o_ref, *, W, G):
    xm = x_ref[0]
    h1 = jnp.dot(xm, w1_ref[...], preferred_element_type=jnp.float32)
    h1 = jnp.maximum(h1 + b1_ref[...], 0.0).astype(jnp.bfloat16)
    h2 = jnp.concatenate(_gconv_bands(h1, W, G, w2_ref, b2_ref), axis=1)
    acc = jnp.dot(h2, w3_ref[...], preferred_element_type=jnp.float32)
    acc = acc + b3_ref[...] + xm.astype(jnp.float32)
    o_ref[0] = jnp.maximum(acc, 0.0).astype(jnp.bfloat16)


def _avg_kernel(x_ref, o_ref):
    x = x_ref[...].astype(jnp.float32)
    o_ref[...] = jnp.sum(x, axis=1) * (1.0 / x.shape[1])


def _matmul(a, w, bias, residual=None, *, relu):
    M, K = a.shape
    N = w.shape[1]
    tm = _pick_tm(M)
    Mp = _round_up(M, tm)
    if Mp != M:
        a = jnp.pad(a, ((0, Mp - M), (0, 0)))
    tn = 512 if N % 512 == 0 else (256 if N % 256 == 0 else 128)
    bias2 = bias.reshape(1, N)
    grid = (Mp // tm, N // tn)
    in_specs = [
        pl.BlockSpec((tm, K), lambda i, j: (i, 0)),
        pl.BlockSpec((K, tn), lambda i, j: (0, j)),
        pl.BlockSpec((1, tn), lambda i, j: (0, j)),
    ]
    args = [a, w, bias2]
    if residual is not None:
        r = residual
        if Mp != M:
            r = jnp.pad(r, ((0, Mp - M), (0, 0)))
        in_specs.append(pl.BlockSpec((tm, tn), lambda i, j: (i, j)))
        args.append(r)
        body = functools.partial(_mm_res_kernel, relu=relu)
    else:
        body = functools.partial(_mm_kernel, relu=relu)
    out = pl.pallas_call(
        body,
        out_shape=jax.ShapeDtypeStruct((Mp, N), jnp.bfloat16),
        grid=grid,
        in_specs=in_specs,
        out_specs=pl.BlockSpec((tm, tn), lambda i, j: (i, j)),
        compiler_params=_cparams("parallel", "parallel"),
    )(*args)
    return out[:M] if Mp != M else out


def _conv1x1(x, w, b, *, relu, stride=1, residual=None):
    if stride > 1:
        B, H, W, C = x.shape
        z = x.reshape(B, H // 2, 2, W // 2, 2, C)
        x = jnp.transpose(z, (0, 2, 4, 1, 3, 5))[:, 0, 0]
    B, H, W, C = x.shape
    out = _matmul(x.reshape(B * H * W, C), w, b, residual=residual, relu=relu)
    return out.reshape(B, H, W, w.shape[1])


def _fused_conv1_gconv(x, w1, b1, w2, b2):
    B, H, W, C = x.shape
    width = w1.shape[1]
    G = width // 128
    wt = w2.reshape(9, G, 128, G, 128)
    gi = jnp.arange(G)
    wk = wt[:, gi, :, gi, :].reshape(G, 9 * 128, 128)
    out = pl.pallas_call(
        functools.partial(_fused_b_kernel, W=W, G=G),
        out_shape=jax.ShapeDtypeStruct((B, H * W, width), jnp.bfloat16),
        grid=(B,),
        in_specs=[
            pl.BlockSpec((1, H * W, C), lambda b: (b, 0, 0)),
            pl.BlockSpec((C, width), lambda b: (0, 0)),
            pl.BlockSpec((1, width), lambda b: (0, 0)),
            pl.BlockSpec((G, 9 * 128, 128), lambda b: (0, 0, 0)),
            pl.BlockSpec((G, 1, 128), lambda b: (0, 0, 0)),
        ],
        out_specs=pl.BlockSpec((1, H * W, width), lambda b: (b, 0, 0)),
        compiler_params=_cparams("parallel"),
    )(x.reshape(B, H * W, C), w1, b1.reshape(1, width), wk,
      b2.reshape(G, 1, 128))
    return out.reshape(B, H, W, width)


def _fused_bottleneck(x, w1, b1, w2, b2, w3, b3):
    B, H, W, C = x.shape
    width = w1.shape[1]
    N3 = w3.shape[1]
    G = width // 128
    wt = w2.reshape(9, G, 128, G, 128)
    gi = jnp.arange(G)
    wk = wt[:, gi, :, gi, :].reshape(G, 9 * 128, 128)
    out = pl.pallas_call(
        functools.partial(_fused_b3_kernel, W=W, G=G),
        out_shape=jax.ShapeDtypeStruct((B, H * W, N3), jnp.bfloat16),
        grid=(B,),
        in_specs=[
            pl.BlockSpec((1, H * W, C), lambda b: (b, 0, 0)),
            pl.BlockSpec((C, width), lambda b: (0, 0)),
            pl.BlockSpec((1, width), lambda b: (0, 0)),
            pl.BlockSpec((G, 9 * 128, 128), lambda b: (0, 0, 0)),
            pl.BlockSpec((G, 1, 128), lambda b: (0, 0, 0)),
            pl.BlockSpec((width, N3), lambda b: (0, 0)),
            pl.BlockSpec((1, N3), lambda b: (0, 0)),
        ],
        out_specs=pl.BlockSpec((1, H * W, N3), lambda b: (b, 0, 0)),
        compiler_params=_cparams("parallel"),
    )(x.reshape(B, H * W, C), w1, b1.reshape(1, width), wk,
      b2.reshape(G, 1, 128), w3, b3.reshape(1, N3))
    return out.reshape(B, H, W, N3)


def _tap_views(xp, Ho, Wo, kh, kw, stride):
    views = []
    for i in range(kh):
        for j in range(kw):
            views.append(xp[:, i:i + stride * (Ho - 1) + 1:stride,
                            j:j + stride * (Wo - 1) + 1:stride, :])
    return views


def _parity_views3x3s2(x, pad_val):
    B, H, W, C = x.shape
    Ho, Wo = H // 2, W // 2
    z = x.reshape(B, Ho, 2, Wo, 2, C)
    zz = jnp.transpose(z, (0, 2, 4, 1, 3, 5))
    views = []
    for p, rs in ((1, 1), (0, 0), (1, 0)):
        for q, cs in ((1, 1), (0, 0), (1, 0)):
            v = zz[:, p, q]
            if rs:
                v = jnp.pad(v, ((0, 0), (1, 0), (0, 0), (0, 0)),
                            constant_values=pad_val)[:, :Ho]
            if cs:
                v = jnp.pad(v, ((0, 0), (0, 0), (1, 0), (0, 0)),
                            constant_values=pad_val)[:, :, :Wo]
            views.append(v)
    return views, Ho, Wo


def _gconv3x3(x, w2, b2, *, stride):
    B, H, W, C = x.shape
    G = C // 128
    if stride == 2:
        views, Ho, Wo = _parity_views3x3s2(x, 0.0)
    else:
        xp = jnp.pad(x, ((0, 0), (1, 1), (1, 1), (0, 0)))
        Ho, Wo = H, W
        views = _tap_views(xp, Ho, Wo, 3, 3, 1)
    M = B * Ho * Wo
    a = jnp.stack(views, axis=0).reshape(9, M, C)
    wt = w2.reshape(9, G, 128, G, 128)
    gi = jnp.arange(G)
    wk = wt[:, gi, :, gi, :].reshape(G, 9 * 128, 128)
    bias = b2.reshape(G, 1, 128)
    tm = _pick_tm(M)
    Mp = _round_up(M, tm)
    if Mp != M:
        a = jnp.pad(a, ((0, 0), (0, Mp - M), (0, 0)))
    out = pl.pallas_call(
        _gconv_kernel,
        out_shape=jax.ShapeDtypeStruct((Mp, C), jnp.bfloat16),
        grid=(Mp // tm, G),
        in_specs=[
            pl.BlockSpec((9, tm, 128), lambda i, j: (0, i, j)),
            pl.BlockSpec((1, 9 * 128, 128), lambda i, j: (j, 0, 0)),
            pl.BlockSpec((1, 1, 128), lambda i, j: (j, 0, 0)),
        ],
        out_specs=pl.BlockSpec((tm, 128), lambda i, j: (i, j)),
        compiler_params=_cparams("parallel", "parallel"),
    )(a, wk, bias)
    if Mp != M:
        out = out[:M]
    return out.reshape(B, Ho, Wo, C)


def _stem(x_nchw, w, b):
    x = jnp.transpose(x_nchw, (0, 2, 3, 1)).astype(jnp.bfloat16)
    B, H, W, C = x.shape
    Ho, Wo = H // 2, W // 2
    xs = jnp.pad(x, ((0, 0), (3, 3), (3, 3), (0, 0)))
    Y = xs.reshape(B, Ho + 3, 2, Wo + 3, 2, C)
    Y = jnp.transpose(Y, (0, 1, 3, 2, 4, 5)).reshape(B, Ho + 3, Wo + 3, 4 * C)
    views = []
    for a_ in range(4):
        for b_ in range(4):
            views.append(Y[:, a_:a_ + Ho, b_:b_ + Wo, :])
    M = B * Ho * Wo
    amat = jnp.stack(views, axis=3).reshape(M, 16 * 4 * C)
    Kp = w.shape[0]
    amat = jnp.pad(amat, ((0, 0), (0, Kp - 16 * 4 * C)))
    idx = []
    for a_ in range(4):
        for b_ in range(4):
            for p in range(2):
                for q in range(2):
                    for c in range(C):
                        i, j = 2 * a_ + p, 2 * b_ + q
                        idx.append(((i * 7 + j) * C + c) if i < 7 and j < 7
                                   else Kp - 1)
    idx += [Kp - 1] * (Kp - len(idx))
    w4 = jnp.take(w, jnp.array(idx), axis=0)
    out = _matmul(amat, w4, b, relu=True)
    return out.reshape(B, Ho, Wo, w.shape[1])


def _maxpool3x3s2(x):
    B, H, W, C = x.shape
    Ho, Wo = H // 2, W // 2
    z = x.reshape(B, Ho, 2, Wo, 2, C)
    zz = jnp.transpose(z, (0, 2, 4, 1, 3, 5))
    out = pl.pallas_call(
        functools.partial(_maxpool_kernel, HW=Ho * Wo, W=Wo),
        out_shape=jax.ShapeDtypeStruct((B, Ho * Wo, C), x.dtype),
        grid=(B,),
        in_specs=[pl.BlockSpec((1, 2, 2, Ho, Wo, C),
                               lambda b: (b, 0, 0, 0, 0, 0))],
        out_specs=pl.BlockSpec((1, Ho * Wo, C), lambda b: (b, 0, 0)),
        compiler_params=_cparams("parallel"),
    )(zz)
    return out.reshape(B, Ho, Wo, C)


def _avgpool(x):
    B, H, W, C = x.shape
    a = x.reshape(B, H * W, C)
    tc = 256
    out = pl.pallas_call(
        _avg_kernel,
        out_shape=jax.ShapeDtypeStruct((B, C), jnp.float32),
        grid=(C // tc,),
        in_specs=[pl.BlockSpec((B, H * W, tc), lambda j: (0, 0, j))],
        out_specs=pl.BlockSpec((B, tc), lambda j: (0, j)),
        compiler_params=_cparams("parallel"),
    )(a)
    return out


def _bottleneck(x, w1, b1, w2, b2, w3, b3, wd, bd, stride):
    if stride == 1 and x.shape[1] >= 7:
        if wd is None:
            return _fused_bottleneck(x, w1, b1, w2, b2, w3, b3)
        out = _fused_conv1_gconv(x, w1, b1, w2, b2)
    else:
        out = _conv1x1(x, w1, b1, relu=True)
        out = _gconv3x3(out, w2, b2, stride=stride)
    if wd is not None:
        identity = _conv1x1(x, wd, bd, relu=False, stride=stride)
    else:
        identity = x
    B, H, W, C = out.shape
    res = identity.reshape(B * H * W, identity.shape[-1])
    out = _matmul(out.reshape(B * H * W, C), w3, b3, residual=res, relu=True)
    return out.reshape(B, H, W, w3.shape[1])


def kernel(stem_w, stem_b, s0_b0_w1, s0_b0_b1, s0_b0_w2, s0_b0_b2, s0_b0_w3, s0_b0_b3, s0_b0_wd, s0_b0_bd, s0_b1_w1, s0_b1_b1, s0_b1_w2, s0_b1_b2, s0_b1_w3, s0_b1_b3, s0_b2_w1, s0_b2_b1, s0_b2_w2, s0_b2_b2, s0_b2_w3, s0_b2_b3, s1_b0_w1, s1_b0_b1, s1_b0_w2, s1_b0_b2, s1_b0_w3, s1_b0_b3, s1_b0_wd, s1_b0_bd, s1_b1_w1, s1_b1_b1, s1_b1_w2, s1_b1_b2, s1_b1_w3, s1_b1_b3, s1_b2_w1, s1_b2_b1, s1_b2_w2, s1_b2_b2, s1_b2_w3, s1_b2_b3, s1_b3_w1, s1_b3_b1, s1_b3_w2, s1_b3_b2, s1_b3_w3, s1_b3_b3, s2_b0_w1, s2_b0_b1, s2_b0_w2, s2_b0_b2, s2_b0_w3, s2_b0_b3, s2_b0_wd, s2_b0_bd, s2_b1_w1, s2_b1_b1, s2_b1_w2, s2_b1_b2, s2_b1_w3, s2_b1_b3, s2_b2_w1, s2_b2_b1, s2_b2_w2, s2_b2_b2, s2_b2_w3, s2_b2_b3, s2_b3_w1, s2_b3_b1, s2_b3_w2, s2_b3_b2, s2_b3_w3, s2_b3_b3, s2_b4_w1, s2_b4_b1, s2_b4_w2, s2_b4_b2, s2_b4_w3, s2_b4_b3, s2_b5_w1, s2_b5_b1, s2_b5_w2, s2_b5_b2, s2_b5_w3, s2_b5_b3, s3_b0_w1, s3_b0_b1, s3_b0_w2, s3_b0_b2, s3_b0_w3, s3_b0_b3, s3_b0_wd, s3_b0_bd, s3_b1_w1, s3_b1_b1, s3_b1_w2, s3_b1_b2, s3_b1_w3, s3_b1_b3, s3_b2_w1, s3_b2_b1, s3_b2_w2, s3_b2_b2, s3_b2_w3, s3_b2_b3, x):
    L = dict(locals())
    out = _stem(x, stem_w, stem_b)
    out = _maxpool3x3s2(out)
    for si, cnt in enumerate(_COUNTS):
        for bi in range(cnt):
            stride = 2 if (bi == 0 and si > 0) else 1
            out = _bottleneck(
                out,
                L[f"s{si}_b{bi}_w1"], L[f"s{si}_b{bi}_b1"],
                L[f"s{si}_b{bi}_w2"], L[f"s{si}_b{bi}_b2"],
                L[f"s{si}_b{bi}_w3"], L[f"s{si}_b{bi}_b3"],
                L.get(f"s{si}_b{bi}_wd"), L.get(f"s{si}_b{bi}_bd"),
                stride,
            )
    pool = _avgpool(out)
    return pool.reshape(pool.shape[0], -1, 1, 1)
